# Initial kernel scaffold; baseline (speedup 1.0000x reference)
import jax
import jax.numpy as jnp
from jax.experimental import pallas as pl


def kernel(x, edge_index, edge_attr, batch, enc_W, enc_b, W1_0, b1_0, W2_0, b2_0, root_0, bias_0, W1_1, b1_1, W2_1, b2_1, root_1, bias_1, out_W, out_b):
    raise NotImplementedError("write your pallas kernel here")


# speedup vs baseline: 3.1342x; 3.1342x over previous
import numpy as np
import jax
import jax.numpy as jnp
from jax import lax
from jax.experimental import pallas as pl
from jax.experimental.pallas import tpu as pltpu
from jax.experimental.pallas import tpu_sc as plsc

H = 16
NC = 2
NS = 16
NW = NC * NS
CH = 128

_F32 = jnp.float32


def _enc_body(x_ref, w_ref, b_ref, o_ref):
    o_ref[...] = (
        jnp.dot(x_ref[...], w_ref[...], preferred_element_type=_F32)
        + b_ref[...]
    )


def _msg_body(ea_ref, xj_ref, w1_ref, b1_ref, r_ref, t_ref, w2p_ref,
              b2m_ref, o_ref):
    ea = ea_ref[...]
    xj = xj_ref[...]
    h1 = jnp.maximum(
        jnp.dot(ea, w1_ref[...], preferred_element_type=_F32) + b1_ref[...],
        0.0)
    z = (jnp.dot(h1, r_ref[...], preferred_element_type=_F32)
         * jnp.dot(xj, t_ref[...], preferred_element_type=_F32))
    o_ref[...] = (
        jnp.dot(z, w2p_ref[...], preferred_element_type=_F32)
        + jnp.dot(xj, b2m_ref[...], preferred_element_type=_F32)
    )


def _upd_body(s_ref, cnt_ref, h_ref, root_ref, bias_ref, o_ref):
    s = s_ref[0] + s_ref[1]
    cnt = cnt_ref[0] + cnt_ref[1]
    inv = 1.0 / jnp.maximum(cnt, 1.0)
    o_ref[...] = jnp.maximum(
        s * inv
        + jnp.dot(h_ref[...], root_ref[...], preferred_element_type=_F32)
        + bias_ref[...],
        0.0)


def _updout_body(s_ref, cnt_ref, h_ref, root_ref, bias_ref, ow_ref, ob_ref,
                 o_ref):
    s = s_ref[0] + s_ref[1]
    cnt = cnt_ref[0] + cnt_ref[1]
    inv = 1.0 / jnp.maximum(cnt, 1.0)
    h2 = jnp.maximum(
        s * inv
        + jnp.dot(h_ref[...], root_ref[...], preferred_element_type=_F32)
        + bias_ref[...],
        0.0)
    o_ref[...] = (
        jnp.dot(h2, ow_ref[...], preferred_element_type=_F32) + ob_ref[...]
    )


def _msg_call(edge_attr, xj, w1, b1, r, t, w2p, b2m, be=2000):
    e = edge_attr.shape[0]
    grid = e // be
    blk_e = pl.BlockSpec((be, H), lambda i: (i, 0))

    def blk_full(a):
        return pl.BlockSpec(a.shape, lambda i: (0,) * a.ndim)

    return pl.pallas_call(
        _msg_body,
        grid=(grid,),
        in_specs=[blk_e, blk_e, blk_full(w1), blk_full(b1), blk_full(r),
                  blk_full(t), blk_full(w2p), blk_full(b2m)],
        out_specs=blk_e,
        out_shape=jax.ShapeDtypeStruct((e, H), _F32),
    )(edge_attr, xj, w1, b1, r, t, w2p, b2m)


_SC_PARAMS = pltpu.CompilerParams(use_tc_tiling_on_sc=False)


def _sc_mesh():
    return plsc.VectorSubcoreMesh(
        core_axis_name="c", subcore_axis_name="s",
        num_cores=NC, num_subcores=NS)


def _make_gather_cnt(n_pad, e):
    n_chunks = e // CH
    per_tile = (n_chunks + NW - 1) // NW
    rows_per_sub = n_pad // NS

    def body(h_hbm, src_hbm, dst_hbm, zeros_hbm, ones_hbm,
             xj_hbm, cnt_hbm, idx_v, idx2_v, rows_v, ones_v, sem, cnt_sh):
        c = lax.axis_index("c")
        s = lax.axis_index("s")
        wid = s * NC + c
        pltpu.sync_copy(ones_hbm, ones_v)
        pltpu.sync_copy(zeros_hbm,
                        cnt_sh.at[pl.ds(s * rows_per_sub, rows_per_sub)])
        plsc.subcore_barrier()

        def step(i, carry):
            cid = wid + i * NW

            @pl.when(cid < n_chunks)
            def _():
                pltpu.sync_copy(src_hbm.at[cid], idx_v)
                pltpu.async_copy(h_hbm.at[idx_v], rows_v, sem).wait()
                pltpu.sync_copy(rows_v, xj_hbm.at[pl.ds(cid * CH, CH)])
                pltpu.sync_copy(dst_hbm.at[cid], idx2_v)
                pltpu.sync_copy(ones_v, cnt_sh.at[idx2_v], add=True)
            return carry

        lax.fori_loop(0, per_tile, step, 0)
        plsc.subcore_barrier()
        pltpu.sync_copy(
            cnt_sh.at[pl.ds(s * rows_per_sub, rows_per_sub)],
            cnt_hbm.at[c, pl.ds(s * rows_per_sub, rows_per_sub)])

    return pl.kernel(
        body,
        out_type=[
            jax.ShapeDtypeStruct((e, H), _F32),
            jax.ShapeDtypeStruct((NC, n_pad, H), _F32),
        ],
        mesh=_sc_mesh(),
        scratch_types=[
            pltpu.VMEM((CH,), jnp.int32),
            pltpu.VMEM((CH,), jnp.int32),
            pltpu.VMEM((CH, H), _F32),
            pltpu.VMEM((CH, H), _F32),
            pltpu.SemaphoreType.DMA,
            pltpu.VMEM_SHARED((n_pad, H), _F32),
        ],
        compiler_params=_SC_PARAMS,
    )


def _make_gather(n_pad, e):
    n_chunks = e // CH
    per_tile = (n_chunks + NW - 1) // NW

    def body(h_hbm, src_hbm, xj_hbm, idx_v, rows_v, sem):
        c = lax.axis_index("c")
        s = lax.axis_index("s")
        wid = s * NC + c

        def step(i, carry):
            cid = wid + i * NW

            @pl.when(cid < n_chunks)
            def _():
                pltpu.sync_copy(src_hbm.at[cid], idx_v)
                pltpu.async_copy(h_hbm.at[idx_v], rows_v, sem).wait()
                pltpu.sync_copy(rows_v, xj_hbm.at[pl.ds(cid * CH, CH)])
            return carry

        lax.fori_loop(0, per_tile, step, 0)

    return pl.kernel(
        body,
        out_type=jax.ShapeDtypeStruct((e, H), _F32),
        mesh=_sc_mesh(),
        scratch_types=[
            pltpu.VMEM((CH,), jnp.int32),
            pltpu.VMEM((CH, H), _F32),
            pltpu.SemaphoreType.DMA,
        ],
        compiler_params=_SC_PARAMS,
    )


def _make_scatter(n_pad, e):
    n_chunks = e // CH
    per_tile = (n_chunks + NW - 1) // NW
    rows_per_sub = n_pad // NS

    def body(msg_hbm, dst_hbm, zeros_hbm, s_hbm, idx_v, rows_v, s_sh):
        c = lax.axis_index("c")
        s = lax.axis_index("s")
        wid = s * NC + c
        pltpu.sync_copy(zeros_hbm,
                        s_sh.at[pl.ds(s * rows_per_sub, rows_per_sub)])
        plsc.subcore_barrier()

        def step(i, carry):
            cid = wid + i * NW

            @pl.when(cid < n_chunks)
            def _():
                pltpu.sync_copy(dst_hbm.at[cid], idx_v)
                pltpu.sync_copy(msg_hbm.at[pl.ds(cid * CH, CH)], rows_v)
                pltpu.sync_copy(rows_v, s_sh.at[idx_v], add=True)
            return carry

        lax.fori_loop(0, per_tile, step, 0)
        plsc.subcore_barrier()
        pltpu.sync_copy(
            s_sh.at[pl.ds(s * rows_per_sub, rows_per_sub)],
            s_hbm.at[c, pl.ds(s * rows_per_sub, rows_per_sub)])

    return pl.kernel(
        body,
        out_type=jax.ShapeDtypeStruct((NC, n_pad, H), _F32),
        mesh=_sc_mesh(),
        scratch_types=[
            pltpu.VMEM((CH,), jnp.int32),
            pltpu.VMEM((CH, H), _F32),
            pltpu.VMEM_SHARED((n_pad, H), _F32),
        ],
        compiler_params=_SC_PARAMS,
    )


def kernel(x, edge_index, edge_attr, batch, enc_W, enc_b,
           W1_0, b1_0, W2_0, b2_0, root_0, bias_0,
           W1_1, b1_1, W2_1, b2_1, root_1, bias_1,
           out_W, out_b):
    n, d_in = x.shape
    e = edge_attr.shape[0]
    d_out = out_W.shape[1]
    n_pad = ((n + NS * 8 - 1) // (NS * 8)) * NS * 8

    src2d = edge_index[0].reshape(e // CH, CH)
    dst2d = edge_index[1].reshape(e // CH, CH)
    zeros = jnp.zeros((n_pad // NS, H), _F32)
    ones = jnp.ones((CH, H), _F32)
    r_mat = jnp.asarray(np.repeat(np.eye(H, dtype=np.float32), H, axis=1))
    t_mat = jnp.asarray(np.tile(np.eye(H, dtype=np.float32), (1, H)))
    x_pad = jnp.concatenate(
        [x, jnp.zeros((n_pad - n, d_in), _F32)], axis=0)

    h0 = pl.pallas_call(
        _enc_body,
        out_shape=jax.ShapeDtypeStruct((n_pad, H), _F32),
    )(x_pad, enc_W, enc_b.reshape(1, H))

    xj1, cnt = _make_gather_cnt(n_pad, e)(h0, src2d, dst2d, zeros, ones)
    msg1 = _msg_call(edge_attr, xj1, W1_0, b1_0.reshape(1, H), r_mat, t_mat,
                     W2_0.reshape(H * H, H), b2_0.reshape(H, H))
    s1 = _make_scatter(n_pad, e)(msg1, dst2d, zeros)
    h1 = pl.pallas_call(
        _upd_body,
        out_shape=jax.ShapeDtypeStruct((n_pad, H), _F32),
    )(s1, cnt, h0, root_0, bias_0.reshape(1, H))

    xj2 = _make_gather(n_pad, e)(h1, src2d)
    msg2 = _msg_call(edge_attr, xj2, W1_1, b1_1.reshape(1, H), r_mat, t_mat,
                     W2_1.reshape(H * H, H), b2_1.reshape(H, H))
    s2 = _make_scatter(n_pad, e)(msg2, dst2d, zeros)
    out = pl.pallas_call(
        _updout_body,
        out_shape=jax.ShapeDtypeStruct((n_pad, d_out), _F32),
    )(s2, cnt, h1, root_1, bias_1.reshape(1, H), out_W,
      out_b.reshape(1, d_out))
    return out[:n]


# speedup vs baseline: 3.7624x; 1.2004x over previous
import numpy as np
import jax
import jax.numpy as jnp
from jax import lax
from jax.experimental import pallas as pl
from jax.experimental.pallas import tpu as pltpu
from jax.experimental.pallas import tpu_sc as plsc

H = 16
NC = 2
NS = 16
NW = NC * NS
CH = 128

_F32 = jnp.float32


def _enc_body(x_ref, w_ref, b_ref, o_ref):
    o_ref[...] = (
        jnp.dot(x_ref[...], w_ref[...], preferred_element_type=_F32)
        + b_ref[...]
    )


def _msg_body(ea_ref, xj_ref, w1_ref, b1_ref, r_ref, t_ref, w2p_ref,
              b2m_ref, o_ref):
    ea = ea_ref[...]
    xj = xj_ref[...]
    h1 = jnp.maximum(
        jnp.dot(ea, w1_ref[...], preferred_element_type=_F32) + b1_ref[...],
        0.0)
    z = (jnp.dot(h1, r_ref[...], preferred_element_type=_F32)
         * jnp.dot(xj, t_ref[...], preferred_element_type=_F32))
    o_ref[...] = (
        jnp.dot(z, w2p_ref[...], preferred_element_type=_F32)
        + jnp.dot(xj, b2m_ref[...], preferred_element_type=_F32)
    )


def _upd_body(s_ref, cnt_ref, h_ref, root_ref, bias_ref, o_ref):
    s = s_ref[0] + s_ref[1]
    cnt = cnt_ref[0] + cnt_ref[1]
    inv = 1.0 / jnp.maximum(cnt, 1.0)
    o_ref[...] = jnp.maximum(
        s * inv
        + jnp.dot(h_ref[...], root_ref[...], preferred_element_type=_F32)
        + bias_ref[...],
        0.0)


def _updout_body(s_ref, cnt_ref, h_ref, root_ref, bias_ref, ow_ref, ob_ref,
                 o_ref):
    s = s_ref[0] + s_ref[1]
    cnt = cnt_ref[0] + cnt_ref[1]
    inv = 1.0 / jnp.maximum(cnt, 1.0)
    h2 = jnp.maximum(
        s * inv
        + jnp.dot(h_ref[...], root_ref[...], preferred_element_type=_F32)
        + bias_ref[...],
        0.0)
    o_ref[...] = (
        jnp.dot(h2, ow_ref[...], preferred_element_type=_F32) + ob_ref[...]
    )


def _msg_call(edge_attr, xj, w1, b1, r, t, w2p, b2m, be=2000):
    e = edge_attr.shape[0]
    grid = e // be
    blk_e = pl.BlockSpec((be, H), lambda i: (i, 0))

    def blk_full(a):
        return pl.BlockSpec(a.shape, lambda i: (0,) * a.ndim)

    return pl.pallas_call(
        _msg_body,
        grid=(grid,),
        in_specs=[blk_e, blk_e, blk_full(w1), blk_full(b1), blk_full(r),
                  blk_full(t), blk_full(w2p), blk_full(b2m)],
        out_specs=blk_e,
        out_shape=jax.ShapeDtypeStruct((e, H), _F32),
    )(edge_attr, xj, w1, b1, r, t, w2p, b2m)


_SC_PARAMS = pltpu.CompilerParams(use_tc_tiling_on_sc=False)


def _sc_mesh():
    return plsc.VectorSubcoreMesh(
        core_axis_name="c", subcore_axis_name="s",
        num_cores=NC, num_subcores=NS)


def _n_valid(w, n_chunks, per_tile):
    nv = n_chunks - w * per_tile
    nv = jnp.maximum(nv, 0)
    return jnp.minimum(nv, per_tile)


def _make_gather_cnt(n_pad, e, e_pad):
    n_chunks = e // CH
    per_tile = (e_pad // CH) // NW
    rows = per_tile * CH
    rps = n_pad // NS

    def body(h_hbm, src_hbm, dst_hbm, zeros_hbm, ones_hbm,
             xj_hbm, cnt_hbm, idx_v, idx2_v, big_v, ones_v, gsem, csem,
             cnt_sh):
        c = lax.axis_index("c")
        s = lax.axis_index("s")
        w = s * NC + c
        nv = _n_valid(w, n_chunks, per_tile)
        pltpu.sync_copy(src_hbm.at[pl.ds(w * per_tile, per_tile)], idx_v)
        pltpu.sync_copy(dst_hbm.at[pl.ds(w * per_tile, per_tile)], idx2_v)
        pltpu.sync_copy(ones_hbm, ones_v)
        pltpu.sync_copy(zeros_hbm, cnt_sh.at[pl.ds(s * rps, rps)])
        plsc.subcore_barrier()

        def fire_g(j, carry):
            pltpu.async_copy(h_hbm.at[idx_v.at[j]],
                             big_v.at[pl.ds(j * CH, CH)], gsem)
            return carry

        lax.fori_loop(0, per_tile, fire_g, 0)

        def fire_c(j, carry):
            pltpu.async_copy(ones_v, cnt_sh.at[idx2_v.at[j]], csem,
                             add=True)
            return carry

        lax.fori_loop(0, nv, fire_c, 0)

        def drain_g(j, carry):
            pltpu.make_async_copy(h_hbm.at[idx_v.at[0]],
                                  big_v.at[pl.ds(0, CH)], gsem).wait()
            return carry

        lax.fori_loop(0, per_tile, drain_g, 0)
        pltpu.sync_copy(big_v, xj_hbm.at[pl.ds(w * rows, rows)])

        def drain_c(j, carry):
            pltpu.make_async_copy(ones_v, cnt_sh.at[idx2_v.at[0]],
                                  csem).wait()
            return carry

        lax.fori_loop(0, nv, drain_c, 0)
        plsc.subcore_barrier()
        pltpu.sync_copy(cnt_sh.at[pl.ds(s * rps, rps)],
                        cnt_hbm.at[c, pl.ds(s * rps, rps)])

    return pl.kernel(
        body,
        out_type=[
            jax.ShapeDtypeStruct((e_pad, H), _F32),
            jax.ShapeDtypeStruct((NC, n_pad, H), _F32),
        ],
        mesh=_sc_mesh(),
        scratch_types=[
            pltpu.VMEM((per_tile, CH), jnp.int32),
            pltpu.VMEM((per_tile, CH), jnp.int32),
            pltpu.VMEM((rows, H), _F32),
            pltpu.VMEM((CH, H), _F32),
            pltpu.SemaphoreType.DMA,
            pltpu.SemaphoreType.DMA,
            pltpu.VMEM_SHARED((n_pad, H), _F32),
        ],
        compiler_params=_SC_PARAMS,
    )


def _make_gather(n_pad, e, e_pad):
    per_tile = (e_pad // CH) // NW
    rows = per_tile * CH

    def body(h_hbm, src_hbm, xj_hbm, idx_v, big_v, gsem):
        c = lax.axis_index("c")
        s = lax.axis_index("s")
        w = s * NC + c
        pltpu.sync_copy(src_hbm.at[pl.ds(w * per_tile, per_tile)], idx_v)

        def fire_g(j, carry):
            pltpu.async_copy(h_hbm.at[idx_v.at[j]],
                             big_v.at[pl.ds(j * CH, CH)], gsem)
            return carry

        lax.fori_loop(0, per_tile, fire_g, 0)

        def drain_g(j, carry):
            pltpu.make_async_copy(h_hbm.at[idx_v.at[0]],
                                  big_v.at[pl.ds(0, CH)], gsem).wait()
            return carry

        lax.fori_loop(0, per_tile, drain_g, 0)
        pltpu.sync_copy(big_v, xj_hbm.at[pl.ds(w * rows, rows)])

    return pl.kernel(
        body,
        out_type=jax.ShapeDtypeStruct((e_pad, H), _F32),
        mesh=_sc_mesh(),
        scratch_types=[
            pltpu.VMEM((per_tile, CH), jnp.int32),
            pltpu.VMEM((rows, H), _F32),
            pltpu.SemaphoreType.DMA,
        ],
        compiler_params=_SC_PARAMS,
    )


def _make_scatter(n_pad, e, e_pad):
    n_chunks = e // CH
    per_tile = (e_pad // CH) // NW
    rows = per_tile * CH
    rps = n_pad // NS

    def body(msg_hbm, dst_hbm, zeros_hbm, s_hbm, idx_v, big_v, ssem, s_sh):
        c = lax.axis_index("c")
        s = lax.axis_index("s")
        w = s * NC + c
        nv = _n_valid(w, n_chunks, per_tile)
        off = jnp.minimum(w * rows, e - rows)
        pltpu.sync_copy(dst_hbm.at[pl.ds(w * per_tile, per_tile)], idx_v)
        pltpu.sync_copy(msg_hbm.at[pl.ds(off, rows)], big_v)
        pltpu.sync_copy(zeros_hbm, s_sh.at[pl.ds(s * rps, rps)])
        plsc.subcore_barrier()

        def fire_s(j, carry):
            loc = w * rows + j * CH - off
            pltpu.async_copy(big_v.at[pl.ds(loc, CH)],
                             s_sh.at[idx_v.at[j]], ssem, add=True)
            return carry

        lax.fori_loop(0, nv, fire_s, 0)

        def drain_s(j, carry):
            pltpu.make_async_copy(big_v.at[pl.ds(0, CH)],
                                  s_sh.at[idx_v.at[0]], ssem).wait()
            return carry

        lax.fori_loop(0, nv, drain_s, 0)
        plsc.subcore_barrier()
        pltpu.sync_copy(s_sh.at[pl.ds(s * rps, rps)],
                        s_hbm.at[c, pl.ds(s * rps, rps)])

    return pl.kernel(
        body,
        out_type=jax.ShapeDtypeStruct((NC, n_pad, H), _F32),
        mesh=_sc_mesh(),
        scratch_types=[
            pltpu.VMEM((per_tile, CH), jnp.int32),
            pltpu.VMEM((rows, H), _F32),
            pltpu.SemaphoreType.DMA,
            pltpu.VMEM_SHARED((n_pad, H), _F32),
        ],
        compiler_params=_SC_PARAMS,
    )


def kernel(x, edge_index, edge_attr, batch, enc_W, enc_b,
           W1_0, b1_0, W2_0, b2_0, root_0, bias_0,
           W1_1, b1_1, W2_1, b2_1, root_1, bias_1,
           out_W, out_b):
    n, d_in = x.shape
    e = edge_attr.shape[0]
    d_out = out_W.shape[1]
    n_pad = ((n + NS * 8 - 1) // (NS * 8)) * NS * 8
    n_chunks = e // CH
    chunks_pad = ((n_chunks + NW - 1) // NW) * NW
    e_pad = chunks_pad * CH

    src2d = jnp.pad(edge_index[0].reshape(n_chunks, CH),
                    ((0, chunks_pad - n_chunks), (0, 0)))
    dst2d = jnp.pad(edge_index[1].reshape(n_chunks, CH),
                    ((0, chunks_pad - n_chunks), (0, 0)))
    zeros = jnp.zeros((n_pad // NS, H), _F32)
    ones = jnp.ones((CH, H), _F32)
    r_mat = jnp.asarray(np.repeat(np.eye(H, dtype=np.float32), H, axis=1))
    t_mat = jnp.asarray(np.tile(np.eye(H, dtype=np.float32), (1, H)))
    x_pad = jnp.concatenate(
        [x, jnp.zeros((n_pad - n, d_in), _F32)], axis=0)

    h0 = pl.pallas_call(
        _enc_body,
        out_shape=jax.ShapeDtypeStruct((n_pad, H), _F32),
    )(x_pad, enc_W, enc_b.reshape(1, H))

    xj1, cnt = _make_gather_cnt(n_pad, e, e_pad)(
        h0, src2d, dst2d, zeros, ones)
    msg1 = _msg_call(edge_attr, xj1, W1_0, b1_0.reshape(1, H), r_mat, t_mat,
                     W2_0.reshape(H * H, H), b2_0.reshape(H, H))
    s1 = _make_scatter(n_pad, e, e_pad)(msg1, dst2d, zeros)
    h1 = pl.pallas_call(
        _upd_body,
        out_shape=jax.ShapeDtypeStruct((n_pad, H), _F32),
    )(s1, cnt, h0, root_0, bias_0.reshape(1, H))

    xj2 = _make_gather(n_pad, e, e_pad)(h1, src2d)
    msg2 = _msg_call(edge_attr, xj2, W1_1, b1_1.reshape(1, H), r_mat, t_mat,
                     W2_1.reshape(H * H, H), b2_1.reshape(H, H))
    s2 = _make_scatter(n_pad, e, e_pad)(msg2, dst2d, zeros)
    out = pl.pallas_call(
        _updout_body,
        out_shape=jax.ShapeDtypeStruct((n_pad, d_out), _F32),
    )(s2, cnt, h1, root_1, bias_1.reshape(1, H), out_W,
      out_b.reshape(1, d_out))
    return out[:n]


# speedup vs baseline: 5.8308x; 1.5497x over previous
import numpy as np
import jax
import jax.numpy as jnp
from jax import lax
from jax.experimental import pallas as pl
from jax.experimental.pallas import tpu as pltpu
from jax.experimental.pallas import tpu_sc as plsc

H = 16
NC = 2
NS = 16
NW = NC * NS
CH = 128

_F32 = jnp.float32


def _enc_body(x_ref, w_ref, b_ref, o_ref):
    o_ref[...] = (
        jnp.dot(x_ref[...], w_ref[...], preferred_element_type=_F32)
        + b_ref[...]
    )


def _msg_body(ea_ref, xj_ref, w1_ref, b1_ref, r_ref, t_ref, w2p_ref,
              b2m_ref, o_ref):
    ea = ea_ref[...]
    xj = xj_ref[...]
    h1 = jnp.maximum(
        jnp.dot(ea, w1_ref[...], preferred_element_type=_F32) + b1_ref[...],
        0.0)
    z = (jnp.dot(h1, r_ref[...], preferred_element_type=_F32)
         * jnp.dot(xj, t_ref[...], preferred_element_type=_F32))
    o_ref[...] = (
        jnp.dot(z, w2p_ref[...], preferred_element_type=_F32)
        + jnp.dot(xj, b2m_ref[...], preferred_element_type=_F32)
    )


def _upd_body(s_ref, cnt_ref, h_ref, root_ref, bias_ref, o_ref):
    s = s_ref[0] + s_ref[1]
    cnt = cnt_ref[0] + cnt_ref[1]
    inv = 1.0 / jnp.maximum(cnt, 1.0)
    o_ref[...] = jnp.maximum(
        s * inv
        + jnp.dot(h_ref[...], root_ref[...], preferred_element_type=_F32)
        + bias_ref[...],
        0.0)


def _updout_body(s_ref, cnt_ref, h_ref, root_ref, bias_ref, ow_ref, ob_ref,
                 o_ref):
    s = s_ref[0] + s_ref[1]
    cnt = cnt_ref[0] + cnt_ref[1]
    inv = 1.0 / jnp.maximum(cnt, 1.0)
    h2 = jnp.maximum(
        s * inv
        + jnp.dot(h_ref[...], root_ref[...], preferred_element_type=_F32)
        + bias_ref[...],
        0.0)
    o_ref[...] = (
        jnp.dot(h2, ow_ref[...], preferred_element_type=_F32) + ob_ref[...]
    )


def _msg_call(ea_p, xj_p, w1, b1, r, t, w2p, b2m, brows=400):
    rows = ea_p.shape[0]
    grid = rows // brows
    blk_e = pl.BlockSpec((brows, 128), lambda i: (i, 0))

    def blk_full(a):
        return pl.BlockSpec(a.shape, lambda i: (0,) * a.ndim)

    return pl.pallas_call(
        _msg_body,
        grid=(grid,),
        in_specs=[blk_e, blk_e, blk_full(w1), blk_full(b1), blk_full(r),
                  blk_full(t), blk_full(w2p), blk_full(b2m)],
        out_specs=blk_e,
        out_shape=jax.ShapeDtypeStruct((rows, 128), _F32),
    )(ea_p, xj_p, w1, b1, r, t, w2p, b2m)


_SC_PARAMS = pltpu.CompilerParams(use_tc_tiling_on_sc=False)


def _sc_mesh():
    return plsc.VectorSubcoreMesh(
        core_axis_name="c", subcore_axis_name="s",
        num_cores=NC, num_subcores=NS)


def _n_valid(w, n_chunks, per_tile):
    nv = n_chunks - w * per_tile
    nv = jnp.maximum(nv, 0)
    return jnp.minimum(nv, per_tile)


def _make_gather_cnt(n_pad, e, e_pad):
    n_chunks = e // CH
    per_tile = (e_pad // CH) // NW
    rows = per_tile * CH
    rps = n_pad // NS

    def body(h_hbm, src_hbm, dst_hbm, zeros_hbm, ones_hbm,
             xj_hbm, cnt_hbm, idx_v, idx2_v, big_v, ones_v, gsem, csem,
             cnt_sh):
        c = lax.axis_index("c")
        s = lax.axis_index("s")
        w = s * NC + c
        nv = _n_valid(w, n_chunks, per_tile)
        pltpu.sync_copy(src_hbm.at[pl.ds(w * per_tile, per_tile)], idx_v)
        pltpu.sync_copy(dst_hbm.at[pl.ds(w * per_tile, per_tile)], idx2_v)
        pltpu.sync_copy(ones_hbm, ones_v)
        pltpu.sync_copy(zeros_hbm, cnt_sh.at[pl.ds(s * rps, rps)])
        plsc.subcore_barrier()

        def fire_g(j, carry):
            pltpu.async_copy(h_hbm.at[idx_v.at[j]],
                             big_v.at[pl.ds(j * CH, CH)], gsem)
            return carry

        lax.fori_loop(0, per_tile, fire_g, 0)

        def fire_c(j, carry):
            pltpu.async_copy(ones_v, cnt_sh.at[idx2_v.at[j]], csem,
                             add=True)
            return carry

        lax.fori_loop(0, nv, fire_c, 0)

        def drain_g(j, carry):
            pltpu.make_async_copy(h_hbm.at[idx_v.at[0]],
                                  big_v.at[pl.ds(0, CH)], gsem).wait()
            return carry

        lax.fori_loop(0, per_tile, drain_g, 0)
        pltpu.sync_copy(big_v, xj_hbm.at[pl.ds(w * rows, rows)])

        def drain_c(j, carry):
            pltpu.make_async_copy(ones_v, cnt_sh.at[idx2_v.at[0]],
                                  csem).wait()
            return carry

        lax.fori_loop(0, nv, drain_c, 0)
        plsc.subcore_barrier()
        pltpu.sync_copy(cnt_sh.at[pl.ds(s * rps, rps)],
                        cnt_hbm.at[c, pl.ds(s * rps, rps)])

    return pl.kernel(
        body,
        out_type=[
            jax.ShapeDtypeStruct((e_pad, H), _F32),
            jax.ShapeDtypeStruct((NC, n_pad, H), _F32),
        ],
        mesh=_sc_mesh(),
        scratch_types=[
            pltpu.VMEM((per_tile, CH), jnp.int32),
            pltpu.VMEM((per_tile, CH), jnp.int32),
            pltpu.VMEM((rows, H), _F32),
            pltpu.VMEM((CH, H), _F32),
            pltpu.SemaphoreType.DMA,
            pltpu.SemaphoreType.DMA,
            pltpu.VMEM_SHARED((n_pad, H), _F32),
        ],
        compiler_params=_SC_PARAMS,
    )


def _make_gather(n_pad, e, e_pad):
    per_tile = (e_pad // CH) // NW
    rows = per_tile * CH

    def body(h_hbm, src_hbm, xj_hbm, idx_v, big_v, gsem):
        c = lax.axis_index("c")
        s = lax.axis_index("s")
        w = s * NC + c
        pltpu.sync_copy(src_hbm.at[pl.ds(w * per_tile, per_tile)], idx_v)

        def fire_g(j, carry):
            pltpu.async_copy(h_hbm.at[idx_v.at[j]],
                             big_v.at[pl.ds(j * CH, CH)], gsem)
            return carry

        lax.fori_loop(0, per_tile, fire_g, 0)

        def drain_g(j, carry):
            pltpu.make_async_copy(h_hbm.at[idx_v.at[0]],
                                  big_v.at[pl.ds(0, CH)], gsem).wait()
            return carry

        lax.fori_loop(0, per_tile, drain_g, 0)
        pltpu.sync_copy(big_v, xj_hbm.at[pl.ds(w * rows, rows)])

    return pl.kernel(
        body,
        out_type=jax.ShapeDtypeStruct((e_pad, H), _F32),
        mesh=_sc_mesh(),
        scratch_types=[
            pltpu.VMEM((per_tile, CH), jnp.int32),
            pltpu.VMEM((rows, H), _F32),
            pltpu.SemaphoreType.DMA,
        ],
        compiler_params=_SC_PARAMS,
    )


def _make_scatter(n_pad, e, e_pad):
    n_chunks = e // CH
    per_tile = (e_pad // CH) // NW
    rows = per_tile * CH
    rps = n_pad // NS

    def body(msg_hbm, dst_hbm, zeros_hbm, s_hbm, idx_v, big_v, ssem, s_sh):
        c = lax.axis_index("c")
        s = lax.axis_index("s")
        w = s * NC + c
        nv = _n_valid(w, n_chunks, per_tile)
        off = jnp.minimum(w * rows, e - rows)
        pltpu.sync_copy(dst_hbm.at[pl.ds(w * per_tile, per_tile)], idx_v)
        pltpu.sync_copy(msg_hbm.at[pl.ds(off, rows)], big_v)
        pltpu.sync_copy(zeros_hbm, s_sh.at[pl.ds(s * rps, rps)])
        plsc.subcore_barrier()

        def fire_s(j, carry):
            loc = w * rows + j * CH - off
            pltpu.async_copy(big_v.at[pl.ds(loc, CH)],
                             s_sh.at[idx_v.at[j]], ssem, add=True)
            return carry

        lax.fori_loop(0, nv, fire_s, 0)

        def drain_s(j, carry):
            pltpu.make_async_copy(big_v.at[pl.ds(0, CH)],
                                  s_sh.at[idx_v.at[0]], ssem).wait()
            return carry

        lax.fori_loop(0, nv, drain_s, 0)
        plsc.subcore_barrier()
        pltpu.sync_copy(s_sh.at[pl.ds(s * rps, rps)],
                        s_hbm.at[c, pl.ds(s * rps, rps)])

    return pl.kernel(
        body,
        out_type=jax.ShapeDtypeStruct((NC, n_pad, H), _F32),
        mesh=_sc_mesh(),
        scratch_types=[
            pltpu.VMEM((per_tile, CH), jnp.int32),
            pltpu.VMEM((rows, H), _F32),
            pltpu.SemaphoreType.DMA,
            pltpu.VMEM_SHARED((n_pad, H), _F32),
        ],
        compiler_params=_SC_PARAMS,
    )


def kernel(x, edge_index, edge_attr, batch, enc_W, enc_b,
           W1_0, b1_0, W2_0, b2_0, root_0, bias_0,
           W1_1, b1_1, W2_1, b2_1, root_1, bias_1,
           out_W, out_b):
    n, d_in = x.shape
    e = edge_attr.shape[0]
    d_out = out_W.shape[1]
    n_pad = ((n + NS * 8 - 1) // (NS * 8)) * NS * 8
    n_chunks = e // CH
    chunks_pad = ((n_chunks + NW - 1) // NW) * NW
    e_pad = chunks_pad * CH

    src2d = jnp.pad(edge_index[0].reshape(n_chunks, CH),
                    ((0, chunks_pad - n_chunks), (0, 0)))
    dst2d = jnp.pad(edge_index[1].reshape(n_chunks, CH),
                    ((0, chunks_pad - n_chunks), (0, 0)))
    zeros = jnp.zeros((n_pad // NS, H), _F32)
    ones = jnp.ones((CH, H), _F32)
    x_pad = jnp.concatenate(
        [x, jnp.zeros((n_pad - n, d_in), _F32)], axis=0)

    ea_p = edge_attr.reshape(e // 8, 8 * H)
    eye8 = jnp.eye(8, dtype=_F32)
    r_mat = jnp.asarray(np.repeat(np.eye(H, dtype=np.float32), H, axis=1))
    t_mat = jnp.asarray(np.tile(np.eye(H, dtype=np.float32), (1, H)))
    r_bd = jnp.kron(eye8, r_mat)
    t_bd = jnp.kron(eye8, t_mat)

    def layer_weights(W1, b1, W2, b2):
        return (jnp.kron(eye8, W1),
                jnp.tile(b1, 8).reshape(1, 8 * H),
                jnp.kron(eye8, W2.reshape(H * H, H)),
                jnp.kron(eye8, b2.reshape(H, H)))

    w1bd_0, b1t_0, w2pbd_0, b2bd_0 = layer_weights(W1_0, b1_0, W2_0, b2_0)
    w1bd_1, b1t_1, w2pbd_1, b2bd_1 = layer_weights(W1_1, b1_1, W2_1, b2_1)

    h0 = pl.pallas_call(
        _enc_body,
        out_shape=jax.ShapeDtypeStruct((n_pad, H), _F32),
    )(x_pad, enc_W, enc_b.reshape(1, H))

    xj1, cnt = _make_gather_cnt(n_pad, e, e_pad)(
        h0, src2d, dst2d, zeros, ones)
    msg1 = _msg_call(ea_p, xj1.reshape(e_pad // 8, 8 * H),
                     w1bd_0, b1t_0, r_bd, t_bd, w2pbd_0, b2bd_0)
    s1 = _make_scatter(n_pad, e, e_pad)(msg1.reshape(e, H), dst2d, zeros)
    h1 = pl.pallas_call(
        _upd_body,
        out_shape=jax.ShapeDtypeStruct((n_pad, H), _F32),
    )(s1, cnt, h0, root_0, bias_0.reshape(1, H))

    xj2 = _make_gather(n_pad, e, e_pad)(h1, src2d)
    msg2 = _msg_call(ea_p, xj2.reshape(e_pad // 8, 8 * H),
                     w1bd_1, b1t_1, r_bd, t_bd, w2pbd_1, b2bd_1)
    s2 = _make_scatter(n_pad, e, e_pad)(msg2.reshape(e, H), dst2d, zeros)
    out = pl.pallas_call(
        _updout_body,
        out_shape=jax.ShapeDtypeStruct((n_pad, d_out), _F32),
    )(s2, cnt, h1, root_1, bias_1.reshape(1, H), out_W,
      out_b.reshape(1, d_out))
    return out[:n]


# speedup vs baseline: 5.9216x; 1.0156x over previous
import numpy as np
import jax
import jax.numpy as jnp
from jax import lax
from jax.experimental import pallas as pl
from jax.experimental.pallas import tpu as pltpu
from jax.experimental.pallas import tpu_sc as plsc

H = 16
NC = 2
NS = 16
NW = NC * NS
CH = 128

_F32 = jnp.float32


def _enc_body(x_ref, w_ref, b_ref, o_ref):
    o_ref[...] = (
        jnp.dot(x_ref[...], w_ref[...], preferred_element_type=_F32)
        + b_ref[...]
    )


def _msg_body(ea_ref, xj_ref, w1_ref, b1_ref, r_ref, t_ref, w2p_ref,
              b2m_ref, o_ref):
    ea = ea_ref[...]
    xj = xj_ref[...]
    h1 = jnp.maximum(
        jnp.dot(ea, w1_ref[...], preferred_element_type=_F32) + b1_ref[...],
        0.0)
    bf = jnp.bfloat16
    z = (jnp.dot(h1.astype(bf), r_ref[...], preferred_element_type=_F32)
         * jnp.dot(xj.astype(bf), t_ref[...], preferred_element_type=_F32))
    o_ref[...] = (
        jnp.dot(z.astype(bf), w2p_ref[...], preferred_element_type=_F32)
        + jnp.dot(xj, b2m_ref[...], preferred_element_type=_F32)
    )


def _upd_body(s_ref, cnt_ref, h_ref, root_ref, bias_ref, o_ref):
    s = s_ref[0] + s_ref[1]
    cnt = cnt_ref[0] + cnt_ref[1]
    inv = 1.0 / jnp.maximum(cnt, 1.0)
    o_ref[...] = jnp.maximum(
        s * inv
        + jnp.dot(h_ref[...], root_ref[...], preferred_element_type=_F32)
        + bias_ref[...],
        0.0)


def _updout_body(s_ref, cnt_ref, h_ref, root_ref, bias_ref, ow_ref, ob_ref,
                 o_ref):
    s = s_ref[0] + s_ref[1]
    cnt = cnt_ref[0] + cnt_ref[1]
    inv = 1.0 / jnp.maximum(cnt, 1.0)
    h2 = jnp.maximum(
        s * inv
        + jnp.dot(h_ref[...], root_ref[...], preferred_element_type=_F32)
        + bias_ref[...],
        0.0)
    o_ref[...] = (
        jnp.dot(h2, ow_ref[...], preferred_element_type=_F32) + ob_ref[...]
    )


def _msg_call(ea_p, xj_p, w1, b1, r, t, w2p, b2m, brows=400):
    rows = ea_p.shape[0]
    grid = rows // brows
    blk_e = pl.BlockSpec((brows, 128), lambda i: (i, 0))

    def blk_full(a):
        return pl.BlockSpec(a.shape, lambda i: (0,) * a.ndim)

    return pl.pallas_call(
        _msg_body,
        grid=(grid,),
        in_specs=[blk_e, blk_e, blk_full(w1), blk_full(b1), blk_full(r),
                  blk_full(t), blk_full(w2p), blk_full(b2m)],
        out_specs=blk_e,
        out_shape=jax.ShapeDtypeStruct((rows, 128), _F32),
    )(ea_p, xj_p, w1, b1, r, t, w2p, b2m)


_SC_PARAMS = pltpu.CompilerParams(use_tc_tiling_on_sc=False)


def _sc_mesh():
    return plsc.VectorSubcoreMesh(
        core_axis_name="c", subcore_axis_name="s",
        num_cores=NC, num_subcores=NS)


def _n_valid(w, n_chunks, per_tile):
    nv = n_chunks - w * per_tile
    nv = jnp.maximum(nv, 0)
    return jnp.minimum(nv, per_tile)


def _make_gather_cnt(n_pad, e, e_pad):
    n_chunks = e // CH
    per_tile = (e_pad // CH) // NW
    rows = per_tile * CH
    rps = n_pad // NS

    def body(h_hbm, src_hbm, dst_hbm, zeros_hbm, ones_hbm,
             xj_hbm, cnt_hbm, idx_v, idx2_v, big_v, ones_v, gsem, csem,
             cnt_sh):
        c = lax.axis_index("c")
        s = lax.axis_index("s")
        w = s * NC + c
        nv = _n_valid(w, n_chunks, per_tile)
        pltpu.sync_copy(src_hbm.at[pl.ds(w * per_tile, per_tile)], idx_v)
        pltpu.sync_copy(dst_hbm.at[pl.ds(w * per_tile, per_tile)], idx2_v)
        pltpu.sync_copy(ones_hbm, ones_v)
        pltpu.sync_copy(zeros_hbm, cnt_sh.at[pl.ds(s * rps, rps)])
        plsc.subcore_barrier()

        def fire_g(j, carry):
            pltpu.async_copy(h_hbm.at[idx_v.at[j]],
                             big_v.at[pl.ds(j * CH, CH)], gsem)
            return carry

        lax.fori_loop(0, per_tile, fire_g, 0)

        def fire_c(j, carry):
            pltpu.async_copy(ones_v, cnt_sh.at[idx2_v.at[j]], csem,
                             add=True)
            return carry

        lax.fori_loop(0, nv, fire_c, 0)

        def drain_g(j, carry):
            pltpu.make_async_copy(h_hbm.at[idx_v.at[0]],
                                  big_v.at[pl.ds(0, CH)], gsem).wait()
            return carry

        lax.fori_loop(0, per_tile, drain_g, 0)
        pltpu.sync_copy(big_v, xj_hbm.at[pl.ds(w * rows, rows)])

        def drain_c(j, carry):
            pltpu.make_async_copy(ones_v, cnt_sh.at[idx2_v.at[0]],
                                  csem).wait()
            return carry

        lax.fori_loop(0, nv, drain_c, 0)
        plsc.subcore_barrier()
        pltpu.sync_copy(cnt_sh.at[pl.ds(s * rps, rps)],
                        cnt_hbm.at[c, pl.ds(s * rps, rps)])

    return pl.kernel(
        body,
        out_type=[
            jax.ShapeDtypeStruct((e_pad, H), _F32),
            jax.ShapeDtypeStruct((NC, n_pad, H), _F32),
        ],
        mesh=_sc_mesh(),
        scratch_types=[
            pltpu.VMEM((per_tile, CH), jnp.int32),
            pltpu.VMEM((per_tile, CH), jnp.int32),
            pltpu.VMEM((rows, H), _F32),
            pltpu.VMEM((CH, H), _F32),
            pltpu.SemaphoreType.DMA,
            pltpu.SemaphoreType.DMA,
            pltpu.VMEM_SHARED((n_pad, H), _F32),
        ],
        compiler_params=_SC_PARAMS,
    )


def _make_gather(n_pad, e, e_pad):
    per_tile = (e_pad // CH) // NW
    rows = per_tile * CH

    def body(h_hbm, src_hbm, xj_hbm, idx_v, big_v, gsem):
        c = lax.axis_index("c")
        s = lax.axis_index("s")
        w = s * NC + c
        pltpu.sync_copy(src_hbm.at[pl.ds(w * per_tile, per_tile)], idx_v)

        def fire_g(j, carry):
            pltpu.async_copy(h_hbm.at[idx_v.at[j]],
                             big_v.at[pl.ds(j * CH, CH)], gsem)
            return carry

        lax.fori_loop(0, per_tile, fire_g, 0)

        def drain_g(j, carry):
            pltpu.make_async_copy(h_hbm.at[idx_v.at[0]],
                                  big_v.at[pl.ds(0, CH)], gsem).wait()
            return carry

        lax.fori_loop(0, per_tile, drain_g, 0)
        pltpu.sync_copy(big_v, xj_hbm.at[pl.ds(w * rows, rows)])

    return pl.kernel(
        body,
        out_type=jax.ShapeDtypeStruct((e_pad, H), _F32),
        mesh=_sc_mesh(),
        scratch_types=[
            pltpu.VMEM((per_tile, CH), jnp.int32),
            pltpu.VMEM((rows, H), _F32),
            pltpu.SemaphoreType.DMA,
        ],
        compiler_params=_SC_PARAMS,
    )


def _make_scatter(n_pad, e, e_pad):
    n_chunks = e // CH
    per_tile = (e_pad // CH) // NW
    rows = per_tile * CH
    rps = n_pad // NS

    def body(msg_hbm, dst_hbm, zeros_hbm, s_hbm, idx_v, big_v, ssem, s_sh):
        c = lax.axis_index("c")
        s = lax.axis_index("s")
        w = s * NC + c
        nv = _n_valid(w, n_chunks, per_tile)
        off = jnp.minimum(w * rows, e - rows)
        pltpu.sync_copy(dst_hbm.at[pl.ds(w * per_tile, per_tile)], idx_v)
        pltpu.sync_copy(msg_hbm.at[pl.ds(off, rows)], big_v)
        pltpu.sync_copy(zeros_hbm, s_sh.at[pl.ds(s * rps, rps)])
        plsc.subcore_barrier()

        def fire_s(j, carry):
            loc = w * rows + j * CH - off
            pltpu.async_copy(big_v.at[pl.ds(loc, CH)],
                             s_sh.at[idx_v.at[j]], ssem, add=True)
            return carry

        lax.fori_loop(0, nv, fire_s, 0)

        def drain_s(j, carry):
            pltpu.make_async_copy(big_v.at[pl.ds(0, CH)],
                                  s_sh.at[idx_v.at[0]], ssem).wait()
            return carry

        lax.fori_loop(0, nv, drain_s, 0)
        plsc.subcore_barrier()
        pltpu.sync_copy(s_sh.at[pl.ds(s * rps, rps)],
                        s_hbm.at[c, pl.ds(s * rps, rps)])

    return pl.kernel(
        body,
        out_type=jax.ShapeDtypeStruct((NC, n_pad, H), _F32),
        mesh=_sc_mesh(),
        scratch_types=[
            pltpu.VMEM((per_tile, CH), jnp.int32),
            pltpu.VMEM((rows, H), _F32),
            pltpu.SemaphoreType.DMA,
            pltpu.VMEM_SHARED((n_pad, H), _F32),
        ],
        compiler_params=_SC_PARAMS,
    )


def kernel(x, edge_index, edge_attr, batch, enc_W, enc_b,
           W1_0, b1_0, W2_0, b2_0, root_0, bias_0,
           W1_1, b1_1, W2_1, b2_1, root_1, bias_1,
           out_W, out_b):
    n, d_in = x.shape
    e = edge_attr.shape[0]
    d_out = out_W.shape[1]
    n_pad = ((n + NS * 8 - 1) // (NS * 8)) * NS * 8
    n_chunks = e // CH
    chunks_pad = ((n_chunks + NW - 1) // NW) * NW
    e_pad = chunks_pad * CH

    src2d = jnp.pad(edge_index[0].reshape(n_chunks, CH),
                    ((0, chunks_pad - n_chunks), (0, 0)))
    dst2d = jnp.pad(edge_index[1].reshape(n_chunks, CH),
                    ((0, chunks_pad - n_chunks), (0, 0)))
    zeros = jnp.zeros((n_pad // NS, H), _F32)
    ones = jnp.ones((CH, H), _F32)
    x_pad = jnp.concatenate(
        [x, jnp.zeros((n_pad - n, d_in), _F32)], axis=0)

    ea_p = edge_attr.reshape(e // 8, 8 * H)
    eye8np = np.eye(8, dtype=np.float32)
    r_mat = np.repeat(np.eye(H, dtype=np.float32), H, axis=1)
    t_mat = np.tile(np.eye(H, dtype=np.float32), (1, H))
    r_bd = jnp.asarray(np.kron(eye8np, r_mat), dtype=jnp.bfloat16)
    t_bd = jnp.asarray(np.kron(eye8np, t_mat), dtype=jnp.bfloat16)
    eye8 = jnp.asarray(eye8np)

    def layer_weights(W1, b1, W2, b2):
        return (jnp.kron(eye8, W1),
                jnp.tile(b1, 8).reshape(1, 8 * H),
                jnp.kron(eye8, W2.reshape(H * H, H)).astype(jnp.bfloat16),
                jnp.kron(eye8, b2.reshape(H, H)))

    w1bd_0, b1t_0, w2pbd_0, b2bd_0 = layer_weights(W1_0, b1_0, W2_0, b2_0)
    w1bd_1, b1t_1, w2pbd_1, b2bd_1 = layer_weights(W1_1, b1_1, W2_1, b2_1)

    h0 = pl.pallas_call(
        _enc_body,
        out_shape=jax.ShapeDtypeStruct((n_pad, H), _F32),
    )(x_pad, enc_W, enc_b.reshape(1, H))

    xj1, cnt = _make_gather_cnt(n_pad, e, e_pad)(
        h0, src2d, dst2d, zeros, ones)
    msg1 = _msg_call(ea_p, xj1.reshape(e_pad // 8, 8 * H),
                     w1bd_0, b1t_0, r_bd, t_bd, w2pbd_0, b2bd_0)
    s1 = _make_scatter(n_pad, e, e_pad)(msg1.reshape(e, H), dst2d, zeros)
    h1 = pl.pallas_call(
        _upd_body,
        out_shape=jax.ShapeDtypeStruct((n_pad, H), _F32),
    )(s1, cnt, h0, root_0, bias_0.reshape(1, H))

    xj2 = _make_gather(n_pad, e, e_pad)(h1, src2d)
    msg2 = _msg_call(ea_p, xj2.reshape(e_pad // 8, 8 * H),
                     w1bd_1, b1t_1, r_bd, t_bd, w2pbd_1, b2bd_1)
    s2 = _make_scatter(n_pad, e, e_pad)(msg2.reshape(e, H), dst2d, zeros)
    out = pl.pallas_call(
        _updout_body,
        out_shape=jax.ShapeDtypeStruct((n_pad, d_out), _F32),
    )(s2, cnt, h1, root_1, bias_1.reshape(1, H), out_W,
      out_b.reshape(1, d_out))
    return out[:n]


# speedup vs baseline: 6.2115x; 1.0490x over previous
import numpy as np
import jax
import jax.numpy as jnp
from jax import lax
from jax.experimental import pallas as pl
from jax.experimental.pallas import tpu as pltpu
from jax.experimental.pallas import tpu_sc as plsc

H = 16
NC = 2
NS = 16
NW = NC * NS
CH = 128

_F32 = jnp.float32


def _enc_body(x_ref, w_ref, b_ref, o_ref):
    o_ref[...] = (
        jnp.dot(x_ref[...], w_ref[...], preferred_element_type=_F32)
        + b_ref[...]
    )


def _msg_body(ea_ref, xj_ref, w1_ref, b1_ref, r_ref, t_ref, w2p_ref,
              b2m_ref, o_ref):
    ea = ea_ref[...]
    xj = xj_ref[...]
    h1 = jnp.maximum(
        jnp.dot(ea, w1_ref[...], preferred_element_type=_F32) + b1_ref[...],
        0.0)
    bf = jnp.bfloat16
    z = (jnp.dot(h1.astype(bf), r_ref[...], preferred_element_type=_F32)
         * jnp.dot(xj.astype(bf), t_ref[...], preferred_element_type=_F32))
    o_ref[...] = (
        jnp.dot(z.astype(bf), w2p_ref[...], preferred_element_type=_F32)
        + jnp.dot(xj, b2m_ref[...], preferred_element_type=_F32)
    )


def _upd_body(s_ref, cnt_ref, h_ref, root_ref, bias_ref, o_ref):
    s = s_ref[0] + s_ref[1]
    cnt = cnt_ref[0] + cnt_ref[1]
    inv = 1.0 / jnp.maximum(cnt, 1.0)
    o_ref[...] = jnp.maximum(
        s * inv
        + jnp.dot(h_ref[...], root_ref[...], preferred_element_type=_F32)
        + bias_ref[...],
        0.0)


def _updout_body(s_ref, cnt_ref, h_ref, root_ref, bias_ref, ow_ref, ob_ref,
                 o_ref):
    s = s_ref[0] + s_ref[1]
    cnt = cnt_ref[0] + cnt_ref[1]
    inv = 1.0 / jnp.maximum(cnt, 1.0)
    h2 = jnp.maximum(
        s * inv
        + jnp.dot(h_ref[...], root_ref[...], preferred_element_type=_F32)
        + bias_ref[...],
        0.0)
    o_ref[...] = (
        jnp.dot(h2, ow_ref[...], preferred_element_type=_F32) + ob_ref[...]
    )


def _msg_call(ea_p, xj_p, w1, b1, r, t, w2p, b2m, brows=1000):
    rows = ea_p.shape[0]
    grid = rows // brows
    blk_e = pl.BlockSpec((brows, 128), lambda i: (i, 0))

    def blk_full(a):
        return pl.BlockSpec(a.shape, lambda i: (0,) * a.ndim)

    return pl.pallas_call(
        _msg_body,
        grid=(grid,),
        in_specs=[blk_e, blk_e, blk_full(w1), blk_full(b1), blk_full(r),
                  blk_full(t), blk_full(w2p), blk_full(b2m)],
        out_specs=blk_e,
        out_shape=jax.ShapeDtypeStruct((rows, 128), _F32),
    )(ea_p, xj_p, w1, b1, r, t, w2p, b2m)


_SC_PARAMS = pltpu.CompilerParams(use_tc_tiling_on_sc=False)


def _sc_mesh():
    return plsc.VectorSubcoreMesh(
        core_axis_name="c", subcore_axis_name="s",
        num_cores=NC, num_subcores=NS)


def _n_valid(w, n_chunks, per_tile):
    nv = n_chunks - w * per_tile
    nv = jnp.maximum(nv, 0)
    return jnp.minimum(nv, per_tile)


def _make_gather_cnt(n_pad, e, e_pad):
    n_chunks = e // CH
    per_tile = (e_pad // CH) // NW
    rows = per_tile * CH
    rps = n_pad // NS

    def body(h_hbm, src_hbm, dst_hbm, zeros_hbm, ones_hbm,
             xj_hbm, cnt_hbm, idx_v, idx2_v, big_v, ones_v, gsem, csem,
             cnt_sh):
        c = lax.axis_index("c")
        s = lax.axis_index("s")
        w = s * NC + c
        nv = _n_valid(w, n_chunks, per_tile)
        pltpu.sync_copy(src_hbm.at[pl.ds(w * per_tile, per_tile)], idx_v)
        pltpu.sync_copy(dst_hbm.at[pl.ds(w * per_tile, per_tile)], idx2_v)
        pltpu.sync_copy(ones_hbm, ones_v)
        pltpu.sync_copy(zeros_hbm, cnt_sh.at[pl.ds(s * rps, rps)])
        plsc.subcore_barrier()

        def fire_g(j, carry):
            pltpu.async_copy(h_hbm.at[idx_v.at[j]],
                             big_v.at[pl.ds(j * CH, CH)], gsem)
            return carry

        lax.fori_loop(0, per_tile, fire_g, 0)

        def fire_c(j, carry):
            pltpu.async_copy(ones_v, cnt_sh.at[idx2_v.at[j]], csem,
                             add=True)
            return carry

        lax.fori_loop(0, nv, fire_c, 0)

        def drain_g(j, carry):
            pltpu.make_async_copy(h_hbm.at[idx_v.at[0]],
                                  big_v.at[pl.ds(0, CH)], gsem).wait()
            return carry

        lax.fori_loop(0, per_tile, drain_g, 0)
        pltpu.sync_copy(big_v, xj_hbm.at[pl.ds(w * rows, rows)])

        def drain_c(j, carry):
            pltpu.make_async_copy(ones_v, cnt_sh.at[idx2_v.at[0]],
                                  csem).wait()
            return carry

        lax.fori_loop(0, nv, drain_c, 0)
        plsc.subcore_barrier()
        pltpu.sync_copy(cnt_sh.at[pl.ds(s * rps, rps)],
                        cnt_hbm.at[c, pl.ds(s * rps, rps)])

    return pl.kernel(
        body,
        out_type=[
            jax.ShapeDtypeStruct((e_pad, H), _F32),
            jax.ShapeDtypeStruct((NC, n_pad, H), _F32),
        ],
        mesh=_sc_mesh(),
        scratch_types=[
            pltpu.VMEM((per_tile, CH), jnp.int32),
            pltpu.VMEM((per_tile, CH), jnp.int32),
            pltpu.VMEM((rows, H), _F32),
            pltpu.VMEM((CH, H), _F32),
            pltpu.SemaphoreType.DMA,
            pltpu.SemaphoreType.DMA,
            pltpu.VMEM_SHARED((n_pad, H), _F32),
        ],
        compiler_params=_SC_PARAMS,
    )


def _make_gather(n_pad, e, e_pad):
    per_tile = (e_pad // CH) // NW
    rows = per_tile * CH

    def body(h_hbm, src_hbm, xj_hbm, idx_v, big_v, gsem):
        c = lax.axis_index("c")
        s = lax.axis_index("s")
        w = s * NC + c
        pltpu.sync_copy(src_hbm.at[pl.ds(w * per_tile, per_tile)], idx_v)

        def fire_g(j, carry):
            pltpu.async_copy(h_hbm.at[idx_v.at[j]],
                             big_v.at[pl.ds(j * CH, CH)], gsem)
            return carry

        lax.fori_loop(0, per_tile, fire_g, 0)

        def drain_g(j, carry):
            pltpu.make_async_copy(h_hbm.at[idx_v.at[0]],
                                  big_v.at[pl.ds(0, CH)], gsem).wait()
            return carry

        lax.fori_loop(0, per_tile, drain_g, 0)
        pltpu.sync_copy(big_v, xj_hbm.at[pl.ds(w * rows, rows)])

    return pl.kernel(
        body,
        out_type=jax.ShapeDtypeStruct((e_pad, H), _F32),
        mesh=_sc_mesh(),
        scratch_types=[
            pltpu.VMEM((per_tile, CH), jnp.int32),
            pltpu.VMEM((rows, H), _F32),
            pltpu.SemaphoreType.DMA,
        ],
        compiler_params=_SC_PARAMS,
    )


def _make_scatter(n_pad, e, e_pad):
    n_chunks = e // CH
    per_tile = (e_pad // CH) // NW
    rows = per_tile * CH
    rps = n_pad // NS

    def body(msg_hbm, dst_hbm, zeros_hbm, s_hbm, idx_v, big_v, ssem, s_sh):
        c = lax.axis_index("c")
        s = lax.axis_index("s")
        w = s * NC + c
        nv = _n_valid(w, n_chunks, per_tile)
        off = jnp.minimum(w * rows, e - rows)
        pltpu.sync_copy(dst_hbm.at[pl.ds(w * per_tile, per_tile)], idx_v)
        pltpu.sync_copy(msg_hbm.at[pl.ds(off, rows)], big_v)
        pltpu.sync_copy(zeros_hbm, s_sh.at[pl.ds(s * rps, rps)])
        plsc.subcore_barrier()

        def fire_s(j, carry):
            loc = w * rows + j * CH - off
            pltpu.async_copy(big_v.at[pl.ds(loc, CH)],
                             s_sh.at[idx_v.at[j]], ssem, add=True)
            return carry

        lax.fori_loop(0, nv, fire_s, 0)

        def drain_s(j, carry):
            pltpu.make_async_copy(big_v.at[pl.ds(0, CH)],
                                  s_sh.at[idx_v.at[0]], ssem).wait()
            return carry

        lax.fori_loop(0, nv, drain_s, 0)
        plsc.subcore_barrier()
        pltpu.sync_copy(s_sh.at[pl.ds(s * rps, rps)],
                        s_hbm.at[c, pl.ds(s * rps, rps)])

    return pl.kernel(
        body,
        out_type=jax.ShapeDtypeStruct((NC, n_pad, H), _F32),
        mesh=_sc_mesh(),
        scratch_types=[
            pltpu.VMEM((per_tile, CH), jnp.int32),
            pltpu.VMEM((rows, H), _F32),
            pltpu.SemaphoreType.DMA,
            pltpu.VMEM_SHARED((n_pad, H), _F32),
        ],
        compiler_params=_SC_PARAMS,
    )


def kernel(x, edge_index, edge_attr, batch, enc_W, enc_b,
           W1_0, b1_0, W2_0, b2_0, root_0, bias_0,
           W1_1, b1_1, W2_1, b2_1, root_1, bias_1,
           out_W, out_b):
    n, d_in = x.shape
    e = edge_attr.shape[0]
    d_out = out_W.shape[1]
    n_pad = ((n + NS * 8 - 1) // (NS * 8)) * NS * 8
    n_chunks = e // CH
    chunks_pad = ((n_chunks + NW - 1) // NW) * NW
    e_pad = chunks_pad * CH

    src2d = jnp.pad(edge_index[0].reshape(n_chunks, CH),
                    ((0, chunks_pad - n_chunks), (0, 0)))
    dst2d = jnp.pad(edge_index[1].reshape(n_chunks, CH),
                    ((0, chunks_pad - n_chunks), (0, 0)))
    zeros = jnp.zeros((n_pad // NS, H), _F32)
    ones = jnp.ones((CH, H), _F32)
    x_pad = jnp.concatenate(
        [x, jnp.zeros((n_pad - n, d_in), _F32)], axis=0)

    ea_p = edge_attr.reshape(e // 8, 8 * H)
    eye8np = np.eye(8, dtype=np.float32)
    r_mat = np.repeat(np.eye(H, dtype=np.float32), H, axis=1)
    t_mat = np.tile(np.eye(H, dtype=np.float32), (1, H))
    r_bd = jnp.asarray(np.kron(eye8np, r_mat), dtype=jnp.bfloat16)
    t_bd = jnp.asarray(np.kron(eye8np, t_mat), dtype=jnp.bfloat16)
    eye8 = jnp.asarray(eye8np)

    def layer_weights(W1, b1, W2, b2):
        return (jnp.kron(eye8, W1),
                jnp.tile(b1, 8).reshape(1, 8 * H),
                jnp.kron(eye8, W2.reshape(H * H, H)).astype(jnp.bfloat16),
                jnp.kron(eye8, b2.reshape(H, H)))

    w1bd_0, b1t_0, w2pbd_0, b2bd_0 = layer_weights(W1_0, b1_0, W2_0, b2_0)
    w1bd_1, b1t_1, w2pbd_1, b2bd_1 = layer_weights(W1_1, b1_1, W2_1, b2_1)

    h0 = pl.pallas_call(
        _enc_body,
        out_shape=jax.ShapeDtypeStruct((n_pad, H), _F32),
    )(x_pad, enc_W, enc_b.reshape(1, H))

    xj1, cnt = _make_gather_cnt(n_pad, e, e_pad)(
        h0, src2d, dst2d, zeros, ones)
    msg1 = _msg_call(ea_p, xj1.reshape(e_pad // 8, 8 * H),
                     w1bd_0, b1t_0, r_bd, t_bd, w2pbd_0, b2bd_0)
    s1 = _make_scatter(n_pad, e, e_pad)(msg1.reshape(e, H), dst2d, zeros)
    h1 = pl.pallas_call(
        _upd_body,
        out_shape=jax.ShapeDtypeStruct((n_pad, H), _F32),
    )(s1, cnt, h0, root_0, bias_0.reshape(1, H))

    xj2 = _make_gather(n_pad, e, e_pad)(h1, src2d)
    msg2 = _msg_call(ea_p, xj2.reshape(e_pad // 8, 8 * H),
                     w1bd_1, b1t_1, r_bd, t_bd, w2pbd_1, b2bd_1)
    s2 = _make_scatter(n_pad, e, e_pad)(msg2.reshape(e, H), dst2d, zeros)
    out = pl.pallas_call(
        _updout_body,
        out_shape=jax.ShapeDtypeStruct((n_pad, d_out), _F32),
    )(s2, cnt, h1, root_1, bias_1.reshape(1, H), out_W,
      out_b.reshape(1, d_out))
    return out[:n]


# speedup vs baseline: 6.3676x; 1.0251x over previous
import numpy as np
import jax
import jax.numpy as jnp
from jax import lax
from jax.experimental import pallas as pl
from jax.experimental.pallas import tpu as pltpu
from jax.experimental.pallas import tpu_sc as plsc

H = 16
NC = 2
NS = 16
NW = NC * NS
CH = 128

_F32 = jnp.float32


def _enc_body(x_ref, w_ref, b_ref, o_ref):
    o_ref[...] = (
        jnp.dot(x_ref[...], w_ref[...], preferred_element_type=_F32)
        + b_ref[...]
    )


def _msg_body(ea_ref, xj_ref, w1_ref, b1_ref, r_ref, t_ref, w2p_ref,
              b2m_ref, o_ref):
    ea = ea_ref[...]
    xj = xj_ref[...]
    h1 = jnp.maximum(
        jnp.dot(ea, w1_ref[...], preferred_element_type=_F32) + b1_ref[...],
        0.0)
    bf = jnp.bfloat16
    z = (jnp.dot(h1.astype(bf), r_ref[...], preferred_element_type=_F32)
         * jnp.dot(xj.astype(bf), t_ref[...], preferred_element_type=_F32))
    o_ref[...] = (
        jnp.dot(z.astype(bf), w2p_ref[...], preferred_element_type=_F32)
        + jnp.dot(xj, b2m_ref[...], preferred_element_type=_F32)
    )


def _upd_body(s_ref, cnt_ref, h_ref, root_ref, bias_ref, o_ref):
    s = s_ref[0] + s_ref[1]
    cnt = cnt_ref[0] + cnt_ref[1]
    inv = 1.0 / jnp.maximum(cnt, 1.0)
    o_ref[...] = jnp.maximum(
        s * inv
        + jnp.dot(h_ref[...], root_ref[...], preferred_element_type=_F32)
        + bias_ref[...],
        0.0)


def _updout_body(s_ref, cnt_ref, h_ref, root_ref, bias_ref, ow_ref, ob_ref,
                 o_ref):
    s = s_ref[0] + s_ref[1]
    cnt = cnt_ref[0] + cnt_ref[1]
    inv = 1.0 / jnp.maximum(cnt, 1.0)
    h2 = jnp.maximum(
        s * inv
        + jnp.dot(h_ref[...], root_ref[...], preferred_element_type=_F32)
        + bias_ref[...],
        0.0)
    o_ref[...] = (
        jnp.dot(h2, ow_ref[...], preferred_element_type=_F32) + ob_ref[...]
    )


def _msg_call(ea_p, xj_p, w1, b1, r, t, w2p, b2m, brows=2000):
    rows = ea_p.shape[0]
    grid = rows // brows
    blk_e = pl.BlockSpec((brows, 128), lambda i: (i, 0))

    def blk_full(a):
        return pl.BlockSpec(a.shape, lambda i: (0,) * a.ndim)

    return pl.pallas_call(
        _msg_body,
        grid=(grid,),
        in_specs=[blk_e, blk_e, blk_full(w1), blk_full(b1), blk_full(r),
                  blk_full(t), blk_full(w2p), blk_full(b2m)],
        out_specs=blk_e,
        out_shape=jax.ShapeDtypeStruct((rows, 128), _F32),
    )(ea_p, xj_p, w1, b1, r, t, w2p, b2m)


_SC_PARAMS = pltpu.CompilerParams(use_tc_tiling_on_sc=False)


def _sc_mesh():
    return plsc.VectorSubcoreMesh(
        core_axis_name="c", subcore_axis_name="s",
        num_cores=NC, num_subcores=NS)


def _n_valid(w, n_chunks, per_tile):
    nv = n_chunks - w * per_tile
    nv = jnp.maximum(nv, 0)
    return jnp.minimum(nv, per_tile)


def _make_gather_cnt(n_pad, e, e_pad):
    n_chunks = e // CH
    per_tile = (e_pad // CH) // NW
    rows = per_tile * CH
    rps = n_pad // NS

    def body(h_hbm, src_hbm, dst_hbm, zeros_hbm, ones_hbm,
             xj_hbm, cnt_hbm, idx_v, idx2_v, big_v, ones_v, gsem, csem,
             cnt_sh):
        c = lax.axis_index("c")
        s = lax.axis_index("s")
        w = s * NC + c
        nv = _n_valid(w, n_chunks, per_tile)
        pltpu.sync_copy(src_hbm.at[pl.ds(w * per_tile, per_tile)], idx_v)
        pltpu.sync_copy(dst_hbm.at[pl.ds(w * per_tile, per_tile)], idx2_v)
        pltpu.sync_copy(ones_hbm, ones_v)
        pltpu.sync_copy(zeros_hbm, cnt_sh.at[pl.ds(s * rps, rps)])
        plsc.subcore_barrier()

        def fire_g(j, carry):
            pltpu.async_copy(h_hbm.at[idx_v.at[j]],
                             big_v.at[pl.ds(j * CH, CH)], gsem)
            return carry

        lax.fori_loop(0, per_tile, fire_g, 0)

        def fire_c(j, carry):
            pltpu.async_copy(ones_v, cnt_sh.at[idx2_v.at[j]], csem,
                             add=True)
            return carry

        lax.fori_loop(0, nv, fire_c, 0)

        def drain_g(j, carry):
            pltpu.make_async_copy(h_hbm.at[idx_v.at[0]],
                                  big_v.at[pl.ds(0, CH)], gsem).wait()
            return carry

        lax.fori_loop(0, per_tile, drain_g, 0)
        pltpu.sync_copy(big_v, xj_hbm.at[pl.ds(w * rows, rows)])

        def drain_c(j, carry):
            pltpu.make_async_copy(ones_v, cnt_sh.at[idx2_v.at[0]],
                                  csem).wait()
            return carry

        lax.fori_loop(0, nv, drain_c, 0)
        plsc.subcore_barrier()
        pltpu.sync_copy(cnt_sh.at[pl.ds(s * rps, rps)],
                        cnt_hbm.at[c, pl.ds(s * rps, rps)])

    return pl.kernel(
        body,
        out_type=[
            jax.ShapeDtypeStruct((e_pad, H), _F32),
            jax.ShapeDtypeStruct((NC, n_pad, H), _F32),
        ],
        mesh=_sc_mesh(),
        scratch_types=[
            pltpu.VMEM((per_tile, CH), jnp.int32),
            pltpu.VMEM((per_tile, CH), jnp.int32),
            pltpu.VMEM((rows, H), _F32),
            pltpu.VMEM((CH, H), _F32),
            pltpu.SemaphoreType.DMA,
            pltpu.SemaphoreType.DMA,
            pltpu.VMEM_SHARED((n_pad, H), _F32),
        ],
        compiler_params=_SC_PARAMS,
    )


def _make_gather(n_pad, e, e_pad):
    per_tile = (e_pad // CH) // NW
    rows = per_tile * CH

    def body(h_hbm, src_hbm, xj_hbm, idx_v, big_v, gsem):
        c = lax.axis_index("c")
        s = lax.axis_index("s")
        w = s * NC + c
        pltpu.sync_copy(src_hbm.at[pl.ds(w * per_tile, per_tile)], idx_v)

        def fire_g(j, carry):
            pltpu.async_copy(h_hbm.at[idx_v.at[j]],
                             big_v.at[pl.ds(j * CH, CH)], gsem)
            return carry

        lax.fori_loop(0, per_tile, fire_g, 0)

        def drain_g(j, carry):
            pltpu.make_async_copy(h_hbm.at[idx_v.at[0]],
                                  big_v.at[pl.ds(0, CH)], gsem).wait()
            return carry

        lax.fori_loop(0, per_tile, drain_g, 0)
        pltpu.sync_copy(big_v, xj_hbm.at[pl.ds(w * rows, rows)])

    return pl.kernel(
        body,
        out_type=jax.ShapeDtypeStruct((e_pad, H), _F32),
        mesh=_sc_mesh(),
        scratch_types=[
            pltpu.VMEM((per_tile, CH), jnp.int32),
            pltpu.VMEM((rows, H), _F32),
            pltpu.SemaphoreType.DMA,
        ],
        compiler_params=_SC_PARAMS,
    )


def _make_scatter(n_pad, e, e_pad):
    n_chunks = e // CH
    per_tile = (e_pad // CH) // NW
    rows = per_tile * CH
    rps = n_pad // NS

    def body(msg_hbm, dst_hbm, zeros_hbm, s_hbm, idx_v, big_v, ssem, s_sh):
        c = lax.axis_index("c")
        s = lax.axis_index("s")
        w = s * NC + c
        nv = _n_valid(w, n_chunks, per_tile)
        off = jnp.minimum(w * rows, e - rows)
        pltpu.sync_copy(dst_hbm.at[pl.ds(w * per_tile, per_tile)], idx_v)
        pltpu.sync_copy(msg_hbm.at[pl.ds(off, rows)], big_v)
        pltpu.sync_copy(zeros_hbm, s_sh.at[pl.ds(s * rps, rps)])
        plsc.subcore_barrier()

        def fire_s(j, carry):
            loc = w * rows + j * CH - off
            pltpu.async_copy(big_v.at[pl.ds(loc, CH)],
                             s_sh.at[idx_v.at[j]], ssem, add=True)
            return carry

        lax.fori_loop(0, nv, fire_s, 0)

        def drain_s(j, carry):
            pltpu.make_async_copy(big_v.at[pl.ds(0, CH)],
                                  s_sh.at[idx_v.at[0]], ssem).wait()
            return carry

        lax.fori_loop(0, nv, drain_s, 0)
        plsc.subcore_barrier()
        pltpu.sync_copy(s_sh.at[pl.ds(s * rps, rps)],
                        s_hbm.at[c, pl.ds(s * rps, rps)])

    return pl.kernel(
        body,
        out_type=jax.ShapeDtypeStruct((NC, n_pad, H), _F32),
        mesh=_sc_mesh(),
        scratch_types=[
            pltpu.VMEM((per_tile, CH), jnp.int32),
            pltpu.VMEM((rows, H), _F32),
            pltpu.SemaphoreType.DMA,
            pltpu.VMEM_SHARED((n_pad, H), _F32),
        ],
        compiler_params=_SC_PARAMS,
    )


def kernel(x, edge_index, edge_attr, batch, enc_W, enc_b,
           W1_0, b1_0, W2_0, b2_0, root_0, bias_0,
           W1_1, b1_1, W2_1, b2_1, root_1, bias_1,
           out_W, out_b):
    n, d_in = x.shape
    e = edge_attr.shape[0]
    d_out = out_W.shape[1]
    n_pad = ((n + NS * 8 - 1) // (NS * 8)) * NS * 8
    n_chunks = e // CH
    chunks_pad = ((n_chunks + NW - 1) // NW) * NW
    e_pad = chunks_pad * CH

    src2d = jnp.pad(edge_index[0].reshape(n_chunks, CH),
                    ((0, chunks_pad - n_chunks), (0, 0)))
    dst2d = jnp.pad(edge_index[1].reshape(n_chunks, CH),
                    ((0, chunks_pad - n_chunks), (0, 0)))
    zeros = jnp.zeros((n_pad // NS, H), _F32)
    ones = jnp.ones((CH, H), _F32)
    x_pad = jnp.concatenate(
        [x, jnp.zeros((n_pad - n, d_in), _F32)], axis=0)

    ea_p = edge_attr.reshape(e // 8, 8 * H)
    eye8np = np.eye(8, dtype=np.float32)
    r_mat = np.repeat(np.eye(H, dtype=np.float32), H, axis=1)
    t_mat = np.tile(np.eye(H, dtype=np.float32), (1, H))
    r_bd = jnp.asarray(np.kron(eye8np, r_mat), dtype=jnp.bfloat16)
    t_bd = jnp.asarray(np.kron(eye8np, t_mat), dtype=jnp.bfloat16)
    eye8 = jnp.asarray(eye8np)

    def layer_weights(W1, b1, W2, b2):
        return (jnp.kron(eye8, W1),
                jnp.tile(b1, 8).reshape(1, 8 * H),
                jnp.kron(eye8, W2.reshape(H * H, H)).astype(jnp.bfloat16),
                jnp.kron(eye8, b2.reshape(H, H)))

    w1bd_0, b1t_0, w2pbd_0, b2bd_0 = layer_weights(W1_0, b1_0, W2_0, b2_0)
    w1bd_1, b1t_1, w2pbd_1, b2bd_1 = layer_weights(W1_1, b1_1, W2_1, b2_1)

    h0 = pl.pallas_call(
        _enc_body,
        out_shape=jax.ShapeDtypeStruct((n_pad, H), _F32),
    )(x_pad, enc_W, enc_b.reshape(1, H))

    xj1, cnt = _make_gather_cnt(n_pad, e, e_pad)(
        h0, src2d, dst2d, zeros, ones)
    msg1 = _msg_call(ea_p, xj1.reshape(e_pad // 8, 8 * H),
                     w1bd_0, b1t_0, r_bd, t_bd, w2pbd_0, b2bd_0)
    s1 = _make_scatter(n_pad, e, e_pad)(msg1.reshape(e, H), dst2d, zeros)
    h1 = pl.pallas_call(
        _upd_body,
        out_shape=jax.ShapeDtypeStruct((n_pad, H), _F32),
    )(s1, cnt, h0, root_0, bias_0.reshape(1, H))

    xj2 = _make_gather(n_pad, e, e_pad)(h1, src2d)
    msg2 = _msg_call(ea_p, xj2.reshape(e_pad // 8, 8 * H),
                     w1bd_1, b1t_1, r_bd, t_bd, w2pbd_1, b2bd_1)
    s2 = _make_scatter(n_pad, e, e_pad)(msg2.reshape(e, H), dst2d, zeros)
    out = pl.pallas_call(
        _updout_body,
        out_shape=jax.ShapeDtypeStruct((n_pad, d_out), _F32),
    )(s2, cnt, h1, root_1, bias_1.reshape(1, H), out_W,
      out_b.reshape(1, d_out))
    return out[:n]


# speedup vs baseline: 6.5298x; 1.0255x over previous
import numpy as np
import jax
import jax.numpy as jnp
from jax import lax
from jax.experimental import pallas as pl
from jax.experimental.pallas import tpu as pltpu
from jax.experimental.pallas import tpu_sc as plsc

H = 16
NC = 2
NS = 16
NW = NC * NS
CH = 128

_F32 = jnp.float32


def _enc_body(x_ref, w_ref, b_ref, o_ref):
    n = x_ref.shape[0]
    o_ref[pl.ds(0, n), :] = (
        jnp.dot(x_ref[...], w_ref[...], preferred_element_type=_F32)
        + b_ref[...]
    )


def _msg_body(ea_ref, xj_ref, w1_ref, b1_ref, r_ref, t_ref, w2p_ref,
              b2m_ref, o_ref):
    ea = ea_ref[...]
    xj = xj_ref[...]
    h1 = jnp.maximum(
        jnp.dot(ea, w1_ref[...], preferred_element_type=_F32) + b1_ref[...],
        0.0)
    bf = jnp.bfloat16
    z = (jnp.dot(h1.astype(bf), r_ref[...], preferred_element_type=_F32)
         * jnp.dot(xj.astype(bf), t_ref[...], preferred_element_type=_F32))
    o_ref[...] = (
        jnp.dot(z.astype(bf), w2p_ref[...], preferred_element_type=_F32)
        + jnp.dot(xj, b2m_ref[...], preferred_element_type=_F32)
    )


def _upd_body(s_ref, cnt_ref, h_ref, root_ref, bias_ref, o_ref):
    s = s_ref[0] + s_ref[1]
    cnt = cnt_ref[0] + cnt_ref[1]
    inv = 1.0 / jnp.maximum(cnt, 1.0)
    o_ref[...] = jnp.maximum(
        s * inv
        + jnp.dot(h_ref[...], root_ref[...], preferred_element_type=_F32)
        + bias_ref[...],
        0.0)


def _updout_body(s_ref, cnt_ref, h_ref, root_ref, bias_ref, ow_ref, ob_ref,
                 o_ref):
    n = o_ref.shape[0]
    s = s_ref[0, pl.ds(0, n)] + s_ref[1, pl.ds(0, n)]
    cnt = cnt_ref[0, pl.ds(0, n)] + cnt_ref[1, pl.ds(0, n)]
    inv = 1.0 / jnp.maximum(cnt, 1.0)
    h2 = jnp.maximum(
        s * inv
        + jnp.dot(h_ref[pl.ds(0, n), :], root_ref[...],
                  preferred_element_type=_F32)
        + bias_ref[...],
        0.0)
    o_ref[...] = (
        jnp.dot(h2, ow_ref[...], preferred_element_type=_F32) + ob_ref[...]
    )


def _msg_call(ea_p, xj_p, w1, b1, r, t, w2p, b2m, brows=2000):
    rows = ea_p.shape[0]
    grid = rows // brows
    blk_e = pl.BlockSpec((brows, 128), lambda i: (i, 0))

    def blk_full(a):
        return pl.BlockSpec(a.shape, lambda i: (0,) * a.ndim)

    return pl.pallas_call(
        _msg_body,
        grid=(grid,),
        in_specs=[blk_e, blk_e, blk_full(w1), blk_full(b1), blk_full(r),
                  blk_full(t), blk_full(w2p), blk_full(b2m)],
        out_specs=blk_e,
        out_shape=jax.ShapeDtypeStruct((rows, 128), _F32),
    )(ea_p, xj_p, w1, b1, r, t, w2p, b2m)


_SC_PARAMS = pltpu.CompilerParams(use_tc_tiling_on_sc=False)


def _sc_mesh():
    return plsc.VectorSubcoreMesh(
        core_axis_name="c", subcore_axis_name="s",
        num_cores=NC, num_subcores=NS)


def _n_valid(w, n_chunks, per_tile):
    nv = n_chunks - w * per_tile
    nv = jnp.maximum(nv, 0)
    return jnp.minimum(nv, per_tile)


def _make_gather_cnt(n_pad, e, e_pad):
    n_chunks = e // CH
    per_tile = (e_pad // CH) // NW
    rows = per_tile * CH
    rps = n_pad // NS

    def body(h_hbm, src_hbm, dst_hbm, zeros_hbm, ones_hbm,
             xj_hbm, cnt_hbm, idx_v, idx2_v, big_v, ones_v, gsem, csem,
             cnt_sh):
        c = lax.axis_index("c")
        s = lax.axis_index("s")
        w = s * NC + c
        nv = _n_valid(w, n_chunks, per_tile)
        pltpu.sync_copy(src_hbm.at[pl.ds(w * per_tile, per_tile)], idx_v)
        pltpu.sync_copy(dst_hbm.at[pl.ds(w * per_tile, per_tile)], idx2_v)
        pltpu.sync_copy(ones_hbm, ones_v)
        pltpu.sync_copy(zeros_hbm, cnt_sh.at[pl.ds(s * rps, rps)])
        plsc.subcore_barrier()

        def fire_g(j, carry):
            pltpu.async_copy(h_hbm.at[idx_v.at[j]],
                             big_v.at[pl.ds(j * CH, CH)], gsem)
            return carry

        lax.fori_loop(0, per_tile, fire_g, 0)

        def fire_c(j, carry):
            pltpu.async_copy(ones_v, cnt_sh.at[idx2_v.at[j]], csem,
                             add=True)
            return carry

        lax.fori_loop(0, nv, fire_c, 0)

        def drain_g(j, carry):
            pltpu.make_async_copy(h_hbm.at[idx_v.at[0]],
                                  big_v.at[pl.ds(0, CH)], gsem).wait()
            return carry

        lax.fori_loop(0, per_tile, drain_g, 0)
        pltpu.sync_copy(big_v, xj_hbm.at[pl.ds(w * rows, rows)])

        def drain_c(j, carry):
            pltpu.make_async_copy(ones_v, cnt_sh.at[idx2_v.at[0]],
                                  csem).wait()
            return carry

        lax.fori_loop(0, nv, drain_c, 0)
        plsc.subcore_barrier()
        pltpu.sync_copy(cnt_sh.at[pl.ds(s * rps, rps)],
                        cnt_hbm.at[c, pl.ds(s * rps, rps)])

    return pl.kernel(
        body,
        out_type=[
            jax.ShapeDtypeStruct((e_pad, H), _F32),
            jax.ShapeDtypeStruct((NC, n_pad, H), _F32),
        ],
        mesh=_sc_mesh(),
        scratch_types=[
            pltpu.VMEM((per_tile, CH), jnp.int32),
            pltpu.VMEM((per_tile, CH), jnp.int32),
            pltpu.VMEM((rows, H), _F32),
            pltpu.VMEM((CH, H), _F32),
            pltpu.SemaphoreType.DMA,
            pltpu.SemaphoreType.DMA,
            pltpu.VMEM_SHARED((n_pad, H), _F32),
        ],
        compiler_params=_SC_PARAMS,
    )


def _make_gather(n_pad, e, e_pad):
    per_tile = (e_pad // CH) // NW
    rows = per_tile * CH

    def body(h_hbm, src_hbm, xj_hbm, idx_v, big_v, gsem):
        c = lax.axis_index("c")
        s = lax.axis_index("s")
        w = s * NC + c
        pltpu.sync_copy(src_hbm.at[pl.ds(w * per_tile, per_tile)], idx_v)

        def fire_g(j, carry):
            pltpu.async_copy(h_hbm.at[idx_v.at[j]],
                             big_v.at[pl.ds(j * CH, CH)], gsem)
            return carry

        lax.fori_loop(0, per_tile, fire_g, 0)

        def drain_g(j, carry):
            pltpu.make_async_copy(h_hbm.at[idx_v.at[0]],
                                  big_v.at[pl.ds(0, CH)], gsem).wait()
            return carry

        lax.fori_loop(0, per_tile, drain_g, 0)
        pltpu.sync_copy(big_v, xj_hbm.at[pl.ds(w * rows, rows)])

    return pl.kernel(
        body,
        out_type=jax.ShapeDtypeStruct((e_pad, H), _F32),
        mesh=_sc_mesh(),
        scratch_types=[
            pltpu.VMEM((per_tile, CH), jnp.int32),
            pltpu.VMEM((rows, H), _F32),
            pltpu.SemaphoreType.DMA,
        ],
        compiler_params=_SC_PARAMS,
    )


def _make_scatter(n_pad, e, e_pad):
    n_chunks = e // CH
    per_tile = (e_pad // CH) // NW
    rows = per_tile * CH
    rps = n_pad // NS

    def body(msg_hbm, dst_hbm, zeros_hbm, s_hbm, idx_v, big_v, ssem, s_sh):
        c = lax.axis_index("c")
        s = lax.axis_index("s")
        w = s * NC + c
        nv = _n_valid(w, n_chunks, per_tile)
        off = jnp.minimum(w * rows, e - rows)
        pltpu.sync_copy(dst_hbm.at[pl.ds(w * per_tile, per_tile)], idx_v)
        pltpu.sync_copy(msg_hbm.at[pl.ds(off, rows)], big_v)
        pltpu.sync_copy(zeros_hbm, s_sh.at[pl.ds(s * rps, rps)])
        plsc.subcore_barrier()

        def fire_s(j, carry):
            loc = w * rows + j * CH - off
            pltpu.async_copy(big_v.at[pl.ds(loc, CH)],
                             s_sh.at[idx_v.at[j]], ssem, add=True)
            return carry

        lax.fori_loop(0, nv, fire_s, 0)

        def drain_s(j, carry):
            pltpu.make_async_copy(big_v.at[pl.ds(0, CH)],
                                  s_sh.at[idx_v.at[0]], ssem).wait()
            return carry

        lax.fori_loop(0, nv, drain_s, 0)
        plsc.subcore_barrier()
        pltpu.sync_copy(s_sh.at[pl.ds(s * rps, rps)],
                        s_hbm.at[c, pl.ds(s * rps, rps)])

    return pl.kernel(
        body,
        out_type=jax.ShapeDtypeStruct((NC, n_pad, H), _F32),
        mesh=_sc_mesh(),
        scratch_types=[
            pltpu.VMEM((per_tile, CH), jnp.int32),
            pltpu.VMEM((rows, H), _F32),
            pltpu.SemaphoreType.DMA,
            pltpu.VMEM_SHARED((n_pad, H), _F32),
        ],
        compiler_params=_SC_PARAMS,
    )


def kernel(x, edge_index, edge_attr, batch, enc_W, enc_b,
           W1_0, b1_0, W2_0, b2_0, root_0, bias_0,
           W1_1, b1_1, W2_1, b2_1, root_1, bias_1,
           out_W, out_b):
    n, d_in = x.shape
    e = edge_attr.shape[0]
    d_out = out_W.shape[1]
    n_pad = ((n + NS * 8 - 1) // (NS * 8)) * NS * 8
    n_chunks = e // CH
    chunks_pad = ((n_chunks + NW - 1) // NW) * NW
    e_pad = chunks_pad * CH

    src2d = jnp.pad(edge_index[0].reshape(n_chunks, CH),
                    ((0, chunks_pad - n_chunks), (0, 0)))
    dst2d = jnp.pad(edge_index[1].reshape(n_chunks, CH),
                    ((0, chunks_pad - n_chunks), (0, 0)))
    zeros = jnp.zeros((n_pad // NS, H), _F32)
    ones = jnp.ones((CH, H), _F32)

    ea_p = edge_attr.reshape(e // 8, 8 * H)
    eye8np = np.eye(8, dtype=np.float32)
    r_mat = np.repeat(np.eye(H, dtype=np.float32), H, axis=1)
    t_mat = np.tile(np.eye(H, dtype=np.float32), (1, H))
    r_bd = jnp.asarray(np.kron(eye8np, r_mat), dtype=jnp.bfloat16)
    t_bd = jnp.asarray(np.kron(eye8np, t_mat), dtype=jnp.bfloat16)
    eye8 = jnp.asarray(eye8np)

    def layer_weights(W1, b1, W2, b2):
        return (jnp.kron(eye8, W1),
                jnp.tile(b1, 8).reshape(1, 8 * H),
                jnp.kron(eye8, W2.reshape(H * H, H)).astype(jnp.bfloat16),
                jnp.kron(eye8, b2.reshape(H, H)))

    w1bd_0, b1t_0, w2pbd_0, b2bd_0 = layer_weights(W1_0, b1_0, W2_0, b2_0)
    w1bd_1, b1t_1, w2pbd_1, b2bd_1 = layer_weights(W1_1, b1_1, W2_1, b2_1)

    h0 = pl.pallas_call(
        _enc_body,
        out_shape=jax.ShapeDtypeStruct((n_pad, H), _F32),
    )(x, enc_W, enc_b.reshape(1, H))

    xj1, cnt = _make_gather_cnt(n_pad, e, e_pad)(
        h0, src2d, dst2d, zeros, ones)
    msg1 = _msg_call(ea_p, xj1.reshape(e_pad // 8, 8 * H),
                     w1bd_0, b1t_0, r_bd, t_bd, w2pbd_0, b2bd_0)
    s1 = _make_scatter(n_pad, e, e_pad)(msg1.reshape(e, H), dst2d, zeros)
    h1 = pl.pallas_call(
        _upd_body,
        out_shape=jax.ShapeDtypeStruct((n_pad, H), _F32),
    )(s1, cnt, h0, root_0, bias_0.reshape(1, H))

    xj2 = _make_gather(n_pad, e, e_pad)(h1, src2d)
    msg2 = _msg_call(ea_p, xj2.reshape(e_pad // 8, 8 * H),
                     w1bd_1, b1t_1, r_bd, t_bd, w2pbd_1, b2bd_1)
    s2 = _make_scatter(n_pad, e, e_pad)(msg2.reshape(e, H), dst2d, zeros)
    out = pl.pallas_call(
        _updout_body,
        out_shape=jax.ShapeDtypeStruct((n, d_out), _F32),
    )(s2, cnt, h1, root_1, bias_1.reshape(1, H), out_W,
      out_b.reshape(1, d_out))
    return out


# speedup vs baseline: 6.5633x; 1.0051x over previous
import numpy as np
import jax
import jax.numpy as jnp
from jax import lax
from jax.experimental import pallas as pl
from jax.experimental.pallas import tpu as pltpu
from jax.experimental.pallas import tpu_sc as plsc

H = 16
NC = 2
NS = 16
NW = NC * NS
CH = 128

_F32 = jnp.float32


def _enc_body(x_ref, w_ref, b_ref, o_ref):
    n = x_ref.shape[0]
    o_ref[pl.ds(0, n), :] = (
        jnp.dot(x_ref[...], w_ref[...], preferred_element_type=_F32)
        + b_ref[...]
    )


def _msg_body(ea_ref, xj_ref, w1_ref, b1_ref, r_ref, t_ref, w2p_ref,
              b2m_ref, o_ref):
    ea = ea_ref[...]
    xj = xj_ref[...]
    h1 = jnp.maximum(
        jnp.dot(ea, w1_ref[0], preferred_element_type=_F32) + b1_ref[0],
        0.0)
    bf = jnp.bfloat16
    z = (jnp.dot(h1.astype(bf), r_ref[...], preferred_element_type=_F32)
         * jnp.dot(xj.astype(bf), t_ref[...], preferred_element_type=_F32))
    o_ref[...] = (
        jnp.dot(z.astype(bf), w2p_ref[0], preferred_element_type=_F32)
        + jnp.dot(xj, b2m_ref[0], preferred_element_type=_F32)
    )


def _upd_body(s_ref, cnt_ref, h_ref, root_ref, bias_ref, o_ref):
    s = s_ref[0] + s_ref[1]
    cnt = cnt_ref[0] + cnt_ref[1]
    inv = 1.0 / jnp.maximum(cnt, 1.0)
    o_ref[...] = jnp.maximum(
        s * inv
        + jnp.dot(h_ref[...], root_ref[...], preferred_element_type=_F32)
        + bias_ref[...],
        0.0)


def _updout_body(s_ref, cnt_ref, h_ref, root_ref, bias_ref, ow_ref, ob_ref,
                 o_ref):
    n = o_ref.shape[0]
    s = s_ref[0, pl.ds(0, n)] + s_ref[1, pl.ds(0, n)]
    cnt = cnt_ref[0, pl.ds(0, n)] + cnt_ref[1, pl.ds(0, n)]
    inv = 1.0 / jnp.maximum(cnt, 1.0)
    h2 = jnp.maximum(
        s * inv
        + jnp.dot(h_ref[pl.ds(0, n), :], root_ref[...],
                  preferred_element_type=_F32)
        + bias_ref[...],
        0.0)
    o_ref[...] = (
        jnp.dot(h2, ow_ref[...], preferred_element_type=_F32) + ob_ref[...]
    )


def _msg_call(layer, ea_p, xj_p, w1, b1, r, t, w2p, b2m, brows=2000):
    rows = ea_p.shape[0]
    grid = rows // brows
    blk_e = pl.BlockSpec((brows, 128), lambda i: (i, 0))

    def blk_full(a):
        return pl.BlockSpec(a.shape, lambda i: (0,) * a.ndim)

    def blk_layer(a):
        return pl.BlockSpec((1,) + a.shape[1:],
                            lambda i: (layer,) + (0,) * (a.ndim - 1))

    return pl.pallas_call(
        _msg_body,
        grid=(grid,),
        in_specs=[blk_e, blk_e, blk_layer(w1), blk_layer(b1), blk_full(r),
                  blk_full(t), blk_layer(w2p), blk_layer(b2m)],
        out_specs=blk_e,
        out_shape=jax.ShapeDtypeStruct((rows, 128), _F32),
    )(ea_p, xj_p, w1, b1, r, t, w2p, b2m)


_SC_PARAMS = pltpu.CompilerParams(use_tc_tiling_on_sc=False)


def _sc_mesh():
    return plsc.VectorSubcoreMesh(
        core_axis_name="c", subcore_axis_name="s",
        num_cores=NC, num_subcores=NS)


def _n_valid(w, n_chunks, per_tile):
    nv = n_chunks - w * per_tile
    nv = jnp.maximum(nv, 0)
    return jnp.minimum(nv, per_tile)


def _make_gather_cnt(n_pad, e, e_pad):
    n_chunks = e // CH
    per_tile = (e_pad // CH) // NW
    rows = per_tile * CH
    rps = n_pad // NS

    def body(h_hbm, src_hbm, dst_hbm, zeros_hbm, ones_hbm,
             xj_hbm, cnt_hbm, idx_v, idx2_v, big_v, ones_v, gsem, csem,
             cnt_sh):
        c = lax.axis_index("c")
        s = lax.axis_index("s")
        w = s * NC + c
        nv = _n_valid(w, n_chunks, per_tile)
        pltpu.sync_copy(src_hbm.at[pl.ds(w * per_tile, per_tile)], idx_v)
        pltpu.sync_copy(dst_hbm.at[pl.ds(w * per_tile, per_tile)], idx2_v)
        pltpu.sync_copy(ones_hbm, ones_v)
        pltpu.sync_copy(zeros_hbm, cnt_sh.at[pl.ds(s * rps, rps)])
        plsc.subcore_barrier()

        def fire_g(j, carry):
            pltpu.async_copy(h_hbm.at[idx_v.at[j]],
                             big_v.at[pl.ds(j * CH, CH)], gsem)
            return carry

        lax.fori_loop(0, per_tile, fire_g, 0)

        def fire_c(j, carry):
            pltpu.async_copy(ones_v, cnt_sh.at[idx2_v.at[j]], csem,
                             add=True)
            return carry

        lax.fori_loop(0, nv, fire_c, 0)

        def drain_g(j, carry):
            pltpu.make_async_copy(h_hbm.at[idx_v.at[0]],
                                  big_v.at[pl.ds(0, CH)], gsem).wait()
            return carry

        lax.fori_loop(0, per_tile, drain_g, 0)
        pltpu.sync_copy(big_v, xj_hbm.at[pl.ds(w * rows, rows)])

        def drain_c(j, carry):
            pltpu.make_async_copy(ones_v, cnt_sh.at[idx2_v.at[0]],
                                  csem).wait()
            return carry

        lax.fori_loop(0, nv, drain_c, 0)
        plsc.subcore_barrier()
        pltpu.sync_copy(cnt_sh.at[pl.ds(s * rps, rps)],
                        cnt_hbm.at[c, pl.ds(s * rps, rps)])

    return pl.kernel(
        body,
        out_type=[
            jax.ShapeDtypeStruct((e_pad, H), _F32),
            jax.ShapeDtypeStruct((NC, n_pad, H), _F32),
        ],
        mesh=_sc_mesh(),
        scratch_types=[
            pltpu.VMEM((per_tile, CH), jnp.int32),
            pltpu.VMEM((per_tile, CH), jnp.int32),
            pltpu.VMEM((rows, H), _F32),
            pltpu.VMEM((CH, H), _F32),
            pltpu.SemaphoreType.DMA,
            pltpu.SemaphoreType.DMA,
            pltpu.VMEM_SHARED((n_pad, H), _F32),
        ],
        compiler_params=_SC_PARAMS,
    )


def _make_gather(n_pad, e, e_pad):
    per_tile = (e_pad // CH) // NW
    rows = per_tile * CH

    def body(h_hbm, src_hbm, xj_hbm, idx_v, big_v, gsem):
        c = lax.axis_index("c")
        s = lax.axis_index("s")
        w = s * NC + c
        pltpu.sync_copy(src_hbm.at[pl.ds(w * per_tile, per_tile)], idx_v)

        def fire_g(j, carry):
            pltpu.async_copy(h_hbm.at[idx_v.at[j]],
                             big_v.at[pl.ds(j * CH, CH)], gsem)
            return carry

        lax.fori_loop(0, per_tile, fire_g, 0)

        def drain_g(j, carry):
            pltpu.make_async_copy(h_hbm.at[idx_v.at[0]],
                                  big_v.at[pl.ds(0, CH)], gsem).wait()
            return carry

        lax.fori_loop(0, per_tile, drain_g, 0)
        pltpu.sync_copy(big_v, xj_hbm.at[pl.ds(w * rows, rows)])

    return pl.kernel(
        body,
        out_type=jax.ShapeDtypeStruct((e_pad, H), _F32),
        mesh=_sc_mesh(),
        scratch_types=[
            pltpu.VMEM((per_tile, CH), jnp.int32),
            pltpu.VMEM((rows, H), _F32),
            pltpu.SemaphoreType.DMA,
        ],
        compiler_params=_SC_PARAMS,
    )


def _make_scatter(n_pad, e, e_pad):
    n_chunks = e // CH
    per_tile = (e_pad // CH) // NW
    rows = per_tile * CH
    rps = n_pad // NS

    def body(msg_hbm, dst_hbm, zeros_hbm, s_hbm, idx_v, big_v, ssem, s_sh):
        c = lax.axis_index("c")
        s = lax.axis_index("s")
        w = s * NC + c
        nv = _n_valid(w, n_chunks, per_tile)
        off = jnp.minimum(w * rows, e - rows)
        pltpu.sync_copy(dst_hbm.at[pl.ds(w * per_tile, per_tile)], idx_v)
        pltpu.sync_copy(msg_hbm.at[pl.ds(off, rows)], big_v)
        pltpu.sync_copy(zeros_hbm, s_sh.at[pl.ds(s * rps, rps)])
        plsc.subcore_barrier()

        def fire_s(j, carry):
            loc = w * rows + j * CH - off
            pltpu.async_copy(big_v.at[pl.ds(loc, CH)],
                             s_sh.at[idx_v.at[j]], ssem, add=True)
            return carry

        lax.fori_loop(0, nv, fire_s, 0)

        def drain_s(j, carry):
            pltpu.make_async_copy(big_v.at[pl.ds(0, CH)],
                                  s_sh.at[idx_v.at[0]], ssem).wait()
            return carry

        lax.fori_loop(0, nv, drain_s, 0)
        plsc.subcore_barrier()
        pltpu.sync_copy(s_sh.at[pl.ds(s * rps, rps)],
                        s_hbm.at[c, pl.ds(s * rps, rps)])

    return pl.kernel(
        body,
        out_type=jax.ShapeDtypeStruct((NC, n_pad, H), _F32),
        mesh=_sc_mesh(),
        scratch_types=[
            pltpu.VMEM((per_tile, CH), jnp.int32),
            pltpu.VMEM((rows, H), _F32),
            pltpu.SemaphoreType.DMA,
            pltpu.VMEM_SHARED((n_pad, H), _F32),
        ],
        compiler_params=_SC_PARAMS,
    )


def kernel(x, edge_index, edge_attr, batch, enc_W, enc_b,
           W1_0, b1_0, W2_0, b2_0, root_0, bias_0,
           W1_1, b1_1, W2_1, b2_1, root_1, bias_1,
           out_W, out_b):
    n, d_in = x.shape
    e = edge_attr.shape[0]
    d_out = out_W.shape[1]
    n_pad = ((n + NS * 8 - 1) // (NS * 8)) * NS * 8
    n_chunks = e // CH
    chunks_pad = ((n_chunks + NW - 1) // NW) * NW
    e_pad = chunks_pad * CH

    src2d = jnp.pad(edge_index[0].reshape(n_chunks, CH),
                    ((0, chunks_pad - n_chunks), (0, 0)))
    dst2d = jnp.pad(edge_index[1].reshape(n_chunks, CH),
                    ((0, chunks_pad - n_chunks), (0, 0)))
    zeros = jnp.zeros((n_pad // NS, H), _F32)
    ones = jnp.ones((CH, H), _F32)

    ea_p = edge_attr.reshape(e // 8, 8 * H)
    eye8np = np.eye(8, dtype=np.float32)
    r_mat = np.repeat(np.eye(H, dtype=np.float32), H, axis=1)
    t_mat = np.tile(np.eye(H, dtype=np.float32), (1, H))
    r_bd = jnp.asarray(np.kron(eye8np, r_mat), dtype=jnp.bfloat16)
    t_bd = jnp.asarray(np.kron(eye8np, t_mat), dtype=jnp.bfloat16)
    eye8 = jnp.asarray(eye8np)

    def kron8(w):
        a, b = w.shape[1], w.shape[2]
        k = jnp.einsum('ij,lab->liajb', eye8, w)
        return k.reshape(2, 8 * a, 8 * b)

    w1bd = kron8(jnp.stack([W1_0, W1_1]))
    w2pbd = kron8(jnp.stack([W2_0.reshape(H * H, H),
                             W2_1.reshape(H * H, H)])).astype(jnp.bfloat16)
    b2bd = kron8(jnp.stack([b2_0.reshape(H, H), b2_1.reshape(H, H)]))
    b1t = jnp.tile(jnp.stack([b1_0, b1_1]), (1, 8)).reshape(2, 1, 8 * H)

    h0 = pl.pallas_call(
        _enc_body,
        out_shape=jax.ShapeDtypeStruct((n_pad, H), _F32),
    )(x, enc_W, enc_b.reshape(1, H))

    xj1, cnt = _make_gather_cnt(n_pad, e, e_pad)(
        h0, src2d, dst2d, zeros, ones)
    msg1 = _msg_call(0, ea_p, xj1.reshape(e_pad // 8, 8 * H),
                     w1bd, b1t, r_bd, t_bd, w2pbd, b2bd)
    s1 = _make_scatter(n_pad, e, e_pad)(msg1.reshape(e, H), dst2d, zeros)
    h1 = pl.pallas_call(
        _upd_body,
        out_shape=jax.ShapeDtypeStruct((n_pad, H), _F32),
    )(s1, cnt, h0, root_0, bias_0.reshape(1, H))

    xj2 = _make_gather(n_pad, e, e_pad)(h1, src2d)
    msg2 = _msg_call(1, ea_p, xj2.reshape(e_pad // 8, 8 * H),
                     w1bd, b1t, r_bd, t_bd, w2pbd, b2bd)
    s2 = _make_scatter(n_pad, e, e_pad)(msg2.reshape(e, H), dst2d, zeros)
    out = pl.pallas_call(
        _updout_body,
        out_shape=jax.ShapeDtypeStruct((n, d_out), _F32),
    )(s2, cnt, h1, root_1, bias_1.reshape(1, H), out_W,
      out_b.reshape(1, d_out))
    return out
